# baseline (speedup 1.0000x reference)
import functools
import math

import jax
import jax.numpy as jnp
from jax.experimental import pallas as pl
from jax.experimental.pallas import tpu as pltpu


def _blk(total, cap):
    b = min(total, cap)
    while b > 8:
        if total % b == 0 and b % 8 == 0:
            return b
        b -= 8
    return total


def _node0_body(x_ref, Win_ref, bin_ref, W1a_ref, W1b_ref, h_ref, A_ref, B_ref):
    h = jnp.dot(x_ref[...], Win_ref[...], preferred_element_type=jnp.float32, precision=jax.lax.Precision.HIGHEST) + bin_ref[...]
    h_ref[...] = h
    A_ref[...] = jnp.dot(h, W1a_ref[...], preferred_element_type=jnp.float32, precision=jax.lax.Precision.HIGHEST)
    B_ref[...] = jnp.dot(h, W1b_ref[...], preferred_element_type=jnp.float32, precision=jax.lax.Precision.HIGHEST)


def _node0(x, W_in, b_in, W1a, W1b):
    N, IN = x.shape
    D = W_in.shape[1]
    bN = _blk(N, 2048)
    grid = (N // bN,)
    out = pl.pallas_call(
        _node0_body,
        grid=grid,
        in_specs=[
            pl.BlockSpec((bN, IN), lambda i: (i, 0)),
            pl.BlockSpec((IN, D), lambda i: (0, 0)),
            pl.BlockSpec((1, D), lambda i: (0, 0)),
            pl.BlockSpec((D, D), lambda i: (0, 0)),
            pl.BlockSpec((D, D), lambda i: (0, 0)),
        ],
        out_specs=[
            pl.BlockSpec((bN, D), lambda i: (i, 0)),
            pl.BlockSpec((bN, D), lambda i: (i, 0)),
            pl.BlockSpec((bN, D), lambda i: (i, 0)),
        ],
        out_shape=[
            jax.ShapeDtypeStruct((N, D), jnp.float32),
            jax.ShapeDtypeStruct((N, D), jnp.float32),
            jax.ShapeDtypeStruct((N, D), jnp.float32),
        ],
    )(x, W_in, b_in.reshape(1, -1), W1a, W1b)
    return out


def _accum_stats(stats_ref, z, i):
    s = jnp.sum(z, axis=0, keepdims=True)
    ss = jnp.sum(z * z, axis=0, keepdims=True)
    blk = jnp.concatenate([s, ss], axis=0)

    @pl.when(i == 0)
    def _():
        stats_ref[...] = blk

    @pl.when(i > 0)
    def _():
        stats_ref[...] += blk


def _p1_body(g_ref, rad_ref, ea_ref, W1e_ref, w1r_ref, b1_ref, stats_ref):
    i = pl.program_id(0)
    z1 = (g_ref[...] + rad_ref[...] * w1r_ref[...]
          + jnp.dot(ea_ref[...], W1e_ref[...], preferred_element_type=jnp.float32, precision=jax.lax.Precision.HIGHEST)
          + b1_ref[...])
    _accum_stats(stats_ref, z1, i)


def _edge_p1(g, rad, ea, W1e, w1r, b1):
    E, D = g.shape
    bE = _blk(E, 4000)
    return pl.pallas_call(
        _p1_body,
        grid=(E // bE,),
        in_specs=[
            pl.BlockSpec((bE, D), lambda i: (i, 0)),
            pl.BlockSpec((bE, 1), lambda i: (i, 0)),
            pl.BlockSpec((bE, 4), lambda i: (i, 0)),
            pl.BlockSpec((4, D), lambda i: (0, 0)),
            pl.BlockSpec((1, D), lambda i: (0, 0)),
            pl.BlockSpec((1, D), lambda i: (0, 0)),
        ],
        out_specs=pl.BlockSpec((2, D), lambda i: (0, 0)),
        out_shape=jax.ShapeDtypeStruct((2, D), jnp.float32),
    )(g, rad, ea, W1e, w1r.reshape(1, -1), b1.reshape(1, -1))


def _p2_body(g_ref, rad_ref, ea_ref, W1e_ref, w1r_ref, b1_ref, sc_ref, sh_ref,
             W2_ref, b2_ref, z2_ref, stats_ref):
    i = pl.program_id(0)
    z1 = (g_ref[...] + rad_ref[...] * w1r_ref[...]
          + jnp.dot(ea_ref[...], W1e_ref[...], preferred_element_type=jnp.float32, precision=jax.lax.Precision.HIGHEST)
          + b1_ref[...])
    m1 = jnp.maximum(z1 * sc_ref[...] + sh_ref[...], 0.0)
    z2 = jnp.dot(m1, W2_ref[...], preferred_element_type=jnp.float32, precision=jax.lax.Precision.HIGHEST) + b2_ref[...]
    z2_ref[...] = z2
    _accum_stats(stats_ref, z2, i)


def _edge_p2(g, rad, ea, W1e, w1r, b1, sc1, sh1, W2, b2):
    E, D = g.shape
    bE = _blk(E, 4000)
    return pl.pallas_call(
        _p2_body,
        grid=(E // bE,),
        in_specs=[
            pl.BlockSpec((bE, D), lambda i: (i, 0)),
            pl.BlockSpec((bE, 1), lambda i: (i, 0)),
            pl.BlockSpec((bE, 4), lambda i: (i, 0)),
            pl.BlockSpec((4, D), lambda i: (0, 0)),
            pl.BlockSpec((1, D), lambda i: (0, 0)),
            pl.BlockSpec((1, D), lambda i: (0, 0)),
            pl.BlockSpec((1, D), lambda i: (0, 0)),
            pl.BlockSpec((1, D), lambda i: (0, 0)),
            pl.BlockSpec((D, D), lambda i: (0, 0)),
            pl.BlockSpec((1, D), lambda i: (0, 0)),
        ],
        out_specs=[
            pl.BlockSpec((bE, D), lambda i: (i, 0)),
            pl.BlockSpec((2, D), lambda i: (0, 0)),
        ],
        out_shape=[
            jax.ShapeDtypeStruct((E, D), jnp.float32),
            jax.ShapeDtypeStruct((2, D), jnp.float32),
        ],
    )(g, rad, ea, W1e, w1r.reshape(1, -1), b1.reshape(1, -1),
      sc1.reshape(1, -1), sh1.reshape(1, -1), W2, b2.reshape(1, -1))


def _p3_body(z2_ref, sc_ref, sh_ref, W3_ref, b3_ref, m_ref, stats_ref):
    i = pl.program_id(0)
    m = jnp.maximum(z2_ref[...] * sc_ref[...] + sh_ref[...], 0.0)
    m_ref[...] = m
    z3 = jnp.dot(m, W3_ref[...], preferred_element_type=jnp.float32, precision=jax.lax.Precision.HIGHEST) + b3_ref[...]
    _accum_stats(stats_ref, z3, i)


def _edge_p3(z2, sc2, sh2, W3, b3):
    E, D = z2.shape
    bE = _blk(E, 4000)
    return pl.pallas_call(
        _p3_body,
        grid=(E // bE,),
        in_specs=[
            pl.BlockSpec((bE, D), lambda i: (i, 0)),
            pl.BlockSpec((1, D), lambda i: (0, 0)),
            pl.BlockSpec((1, D), lambda i: (0, 0)),
            pl.BlockSpec((D, D), lambda i: (0, 0)),
            pl.BlockSpec((1, D), lambda i: (0, 0)),
        ],
        out_specs=[
            pl.BlockSpec((bE, D), lambda i: (i, 0)),
            pl.BlockSpec((2, D), lambda i: (0, 0)),
        ],
        out_shape=[
            jax.ShapeDtypeStruct((E, D), jnp.float32),
            jax.ShapeDtypeStruct((2, D), jnp.float32),
        ],
    )(z2, sc2.reshape(1, -1), sh2.reshape(1, -1), W3, b3.reshape(1, -1))


def _p4_body(m_ref, pd_ref, sc_ref, sh_ref, W3_ref, b3_ref, W4_ref, b4_ref, pu_ref):
    z3 = jnp.dot(m_ref[...], W3_ref[...], preferred_element_type=jnp.float32, precision=jax.lax.Precision.HIGHEST) + b3_ref[...]
    pm1 = jnp.maximum(z3 * sc_ref[...] + sh_ref[...], 0.0)
    pm = jnp.dot(pm1, W4_ref[...], preferred_element_type=jnp.float32, precision=jax.lax.Precision.HIGHEST) + b4_ref[...]
    pu_ref[...] = pd_ref[...] * pm


def _edge_p4(m, pd, sc3, sh3, W3, b3, W4, b4):
    E, D = m.shape
    bE = _blk(E, 4000)
    return pl.pallas_call(
        _p4_body,
        grid=(E // bE,),
        in_specs=[
            pl.BlockSpec((bE, D), lambda i: (i, 0)),
            pl.BlockSpec((bE, 4), lambda i: (i, 0)),
            pl.BlockSpec((1, D), lambda i: (0, 0)),
            pl.BlockSpec((1, D), lambda i: (0, 0)),
            pl.BlockSpec((D, D), lambda i: (0, 0)),
            pl.BlockSpec((1, D), lambda i: (0, 0)),
            pl.BlockSpec((D, 1), lambda i: (0, 0)),
            pl.BlockSpec((1, 1), lambda i: (0, 0)),
        ],
        out_specs=pl.BlockSpec((bE, 4), lambda i: (i, 0)),
        out_shape=jax.ShapeDtypeStruct((E, 4), jnp.float32),
    )(m, pd, sc3.reshape(1, -1), sh3.reshape(1, -1), W3, b3.reshape(1, -1),
      W4, b4.reshape(1, 1))


def _u1_body(h_ref, ms_ref, Wa_ref, Wb_ref, b_ref, u1_ref, stats_ref):
    i = pl.program_id(0)
    u1 = (jnp.dot(h_ref[...], Wa_ref[...], preferred_element_type=jnp.float32, precision=jax.lax.Precision.HIGHEST)
          + jnp.dot(ms_ref[...], Wb_ref[...], preferred_element_type=jnp.float32, precision=jax.lax.Precision.HIGHEST)
          + b_ref[...])
    u1_ref[...] = u1
    _accum_stats(stats_ref, u1, i)


def _node_u1(h, msum, Wa, Wb, b):
    N, D = h.shape
    bN = _blk(N, 2048)
    return pl.pallas_call(
        _u1_body,
        grid=(N // bN,),
        in_specs=[
            pl.BlockSpec((bN, D), lambda i: (i, 0)),
            pl.BlockSpec((bN, D), lambda i: (i, 0)),
            pl.BlockSpec((D, D), lambda i: (0, 0)),
            pl.BlockSpec((D, D), lambda i: (0, 0)),
            pl.BlockSpec((1, D), lambda i: (0, 0)),
        ],
        out_specs=[
            pl.BlockSpec((bN, D), lambda i: (i, 0)),
            pl.BlockSpec((2, D), lambda i: (0, 0)),
        ],
        out_shape=[
            jax.ShapeDtypeStruct((N, D), jnp.float32),
            jax.ShapeDtypeStruct((2, D), jnp.float32),
        ],
    )(h, msum, Wa, Wb, b.reshape(1, -1))


def _u2_body(u1_ref, sc_ref, sh_ref, W_ref, b_ref, v_ref, stats_ref):
    i = pl.program_id(0)
    r = jnp.maximum(u1_ref[...] * sc_ref[...] + sh_ref[...], 0.0)
    v = jnp.dot(r, W_ref[...], preferred_element_type=jnp.float32, precision=jax.lax.Precision.HIGHEST) + b_ref[...]
    v_ref[...] = v
    _accum_stats(stats_ref, v, i)


def _node_u2(u1, scu, shu, W, b):
    N, D = u1.shape
    bN = _blk(N, 2048)
    return pl.pallas_call(
        _u2_body,
        grid=(N // bN,),
        in_specs=[
            pl.BlockSpec((bN, D), lambda i: (i, 0)),
            pl.BlockSpec((1, D), lambda i: (0, 0)),
            pl.BlockSpec((1, D), lambda i: (0, 0)),
            pl.BlockSpec((D, D), lambda i: (0, 0)),
            pl.BlockSpec((1, D), lambda i: (0, 0)),
        ],
        out_specs=[
            pl.BlockSpec((bN, D), lambda i: (i, 0)),
            pl.BlockSpec((2, D), lambda i: (0, 0)),
        ],
        out_shape=[
            jax.ShapeDtypeStruct((N, D), jnp.float32),
            jax.ShapeDtypeStruct((2, D), jnp.float32),
        ],
    )(u1, scu.reshape(1, -1), shu.reshape(1, -1), W, b.reshape(1, -1))


def _u3_body(h_ref, v_ref, sc_ref, sh_ref, psum_ref, deg_ref, posq_ref,
             W1a_ref, W1b_ref, h2_ref, pq2_ref, A_ref, B_ref):
    h2 = h_ref[...] + jnp.maximum(v_ref[...] * sc_ref[...] + sh_ref[...], 0.0)
    h2_ref[...] = h2
    pq2_ref[...] = posq_ref[...] + psum_ref[...] / jnp.maximum(deg_ref[...], 1.0)
    A_ref[...] = jnp.dot(h2, W1a_ref[...], preferred_element_type=jnp.float32, precision=jax.lax.Precision.HIGHEST)
    B_ref[...] = jnp.dot(h2, W1b_ref[...], preferred_element_type=jnp.float32, precision=jax.lax.Precision.HIGHEST)


def _node_u3(h, v, scv, shv, psum, deg, posq, W1a_next, W1b_next):
    N, D = h.shape
    bN = _blk(N, 2048)
    return pl.pallas_call(
        _u3_body,
        grid=(N // bN,),
        in_specs=[
            pl.BlockSpec((bN, D), lambda i: (i, 0)),
            pl.BlockSpec((bN, D), lambda i: (i, 0)),
            pl.BlockSpec((1, D), lambda i: (0, 0)),
            pl.BlockSpec((1, D), lambda i: (0, 0)),
            pl.BlockSpec((bN, 4), lambda i: (i, 0)),
            pl.BlockSpec((bN, 1), lambda i: (i, 0)),
            pl.BlockSpec((bN, 4), lambda i: (i, 0)),
            pl.BlockSpec((D, D), lambda i: (0, 0)),
            pl.BlockSpec((D, D), lambda i: (0, 0)),
        ],
        out_specs=[
            pl.BlockSpec((bN, D), lambda i: (i, 0)),
            pl.BlockSpec((bN, 4), lambda i: (i, 0)),
            pl.BlockSpec((bN, D), lambda i: (i, 0)),
            pl.BlockSpec((bN, D), lambda i: (i, 0)),
        ],
        out_shape=[
            jax.ShapeDtypeStruct((N, D), jnp.float32),
            jax.ShapeDtypeStruct((N, 4), jnp.float32),
            jax.ShapeDtypeStruct((N, D), jnp.float32),
            jax.ShapeDtypeStruct((N, D), jnp.float32),
        ],
    )(h, v, scv.reshape(1, -1), shv.reshape(1, -1), psum, deg, posq,
      W1a_next, W1b_next)


def _pred_body(h_ref, wp_ref, bp_ref, out_ref, acc_ref):
    i = pl.program_id(0)

    @pl.when(i == 0)
    def _():
        acc_ref[...] = jnp.zeros_like(acc_ref)

    acc_ref[...] += jnp.sum(h_ref[...], axis=0, keepdims=True)

    @pl.when(i == pl.num_programs(0) - 1)
    def _():
        n = pl.num_programs(0) * h_ref.shape[0]
        pooled = acc_ref[...] / jnp.float32(n)
        out_ref[...] = jnp.dot(pooled, wp_ref[...], preferred_element_type=jnp.float32, precision=jax.lax.Precision.HIGHEST) + bp_ref[...]


def _pred(h, W_pred, b_pred):
    N, D = h.shape
    bN = _blk(N, 2048)
    out = pl.pallas_call(
        _pred_body,
        grid=(N // bN,),
        in_specs=[
            pl.BlockSpec((bN, D), lambda i: (i, 0)),
            pl.BlockSpec((D, 1), lambda i: (0, 0)),
            pl.BlockSpec((1, 1), lambda i: (0, 0)),
        ],
        out_specs=pl.BlockSpec((1, 1), lambda i: (0, 0)),
        out_shape=jax.ShapeDtypeStruct((1, 1), jnp.float32),
        scratch_shapes=[pltpu.VMEM((1, D), jnp.float32)],
    )(h, W_pred, b_pred.reshape(1, 1))
    return out.reshape(-1)


def _gather_stage(A, B, posq, dst, src):
    g = jnp.take(A, dst, axis=0) + jnp.take(B, src, axis=0)
    pd = jnp.take(posq, dst, axis=0) - jnp.take(posq, src, axis=0)
    rad = jnp.sum(pd * pd, axis=1, keepdims=True)
    return g, pd, rad


def _scatter_stage(m, pu, dst, N):
    msum = jnp.zeros((N, m.shape[1]), jnp.float32).at[dst].add(m)
    psum = jnp.zeros((N, 4), jnp.float32).at[dst].add(pu)
    return msum, psum


def _degrees(dst, N):
    return jnp.zeros((N, 1), jnp.float32).at[dst, 0].add(1.0)


def _bn_coeffs(stats, count, gamma, beta):
    s, ss = stats[0], stats[1]
    mean = s / count
    var = ss / count - mean * mean
    sc = gamma / jnp.sqrt(var + 1e-5)
    sh = beta - mean * sc
    return sc, sh


def kernel(W_in, b_in, l0_msg_W1, l0_msg_b1, l0_msg_g1, l0_msg_be1, l0_msg_W2, l0_msg_b2, l0_msg_g2, l0_msg_be2, l0_pos_W1, l0_pos_b1, l0_pos_g1, l0_pos_be1, l0_pos_W2, l0_pos_b2, l0_upd_W1, l0_upd_b1, l0_upd_g1, l0_upd_be1, l0_upd_W2, l0_upd_b2, l0_upd_g2, l0_upd_be2, l1_msg_W1, l1_msg_b1, l1_msg_g1, l1_msg_be1, l1_msg_W2, l1_msg_b2, l1_msg_g2, l1_msg_be2, l1_pos_W1, l1_pos_b1, l1_pos_g1, l1_pos_be1, l1_pos_W2, l1_pos_b2, l1_upd_W1, l1_upd_b1, l1_upd_g1, l1_upd_be1, l1_upd_W2, l1_upd_b2, l1_upd_g2, l1_upd_be2, W_pred, b_pred, x, pos, edge_index, edge_attr, batch):
    N = x.shape[0]
    E = edge_index.shape[1]
    D = W_in.shape[1]
    src = edge_index[0]
    dst = edge_index[1]
    posq = jnp.pad(pos, ((0, 0), (0, 1)))

    L = [
        dict(msg_W1=l0_msg_W1, msg_b1=l0_msg_b1, msg_g1=l0_msg_g1, msg_be1=l0_msg_be1,
             msg_W2=l0_msg_W2, msg_b2=l0_msg_b2, msg_g2=l0_msg_g2, msg_be2=l0_msg_be2,
             pos_W1=l0_pos_W1, pos_b1=l0_pos_b1, pos_g1=l0_pos_g1, pos_be1=l0_pos_be1,
             pos_W2=l0_pos_W2, pos_b2=l0_pos_b2,
             upd_W1=l0_upd_W1, upd_b1=l0_upd_b1, upd_g1=l0_upd_g1, upd_be1=l0_upd_be1,
             upd_W2=l0_upd_W2, upd_b2=l0_upd_b2, upd_g2=l0_upd_g2, upd_be2=l0_upd_be2),
        dict(msg_W1=l1_msg_W1, msg_b1=l1_msg_b1, msg_g1=l1_msg_g1, msg_be1=l1_msg_be1,
             msg_W2=l1_msg_W2, msg_b2=l1_msg_b2, msg_g2=l1_msg_g2, msg_be2=l1_msg_be2,
             pos_W1=l1_pos_W1, pos_b1=l1_pos_b1, pos_g1=l1_pos_g1, pos_be1=l1_pos_be1,
             pos_W2=l1_pos_W2, pos_b2=l1_pos_b2,
             upd_W1=l1_upd_W1, upd_b1=l1_upd_b1, upd_g1=l1_upd_g1, upd_be1=l1_upd_be1,
             upd_W2=l1_upd_W2, upd_b2=l1_upd_b2, upd_g2=l1_upd_g2, upd_be2=l1_upd_be2),
    ]

    deg = _degrees(dst, N)

    W1 = L[0]['msg_W1']
    h, A, B = _node0(x, W_in, b_in, W1[:D], W1[D:2 * D])

    for l in range(2):
        p = L[l]
        W1 = p['msg_W1']
        w1r = W1[2 * D]
        W1e = W1[2 * D + 1:]

        g, pd, rad = _gather_stage(A, B, posq, dst, src)

        stats1 = _edge_p1(g, rad, edge_attr, W1e, w1r, p['msg_b1'])
        sc1, sh1 = _bn_coeffs(stats1, E, p['msg_g1'], p['msg_be1'])

        z2, stats2 = _edge_p2(g, rad, edge_attr, W1e, w1r, p['msg_b1'],
                              sc1, sh1, p['msg_W2'], p['msg_b2'])
        sc2, sh2 = _bn_coeffs(stats2, E, p['msg_g2'], p['msg_be2'])

        m, stats3 = _edge_p3(z2, sc2, sh2, p['pos_W1'], p['pos_b1'])
        sc3, sh3 = _bn_coeffs(stats3, E, p['pos_g1'], p['pos_be1'])

        pu = _edge_p4(m, pd, sc3, sh3, p['pos_W1'], p['pos_b1'],
                      p['pos_W2'], p['pos_b2'])

        msum, psum = _scatter_stage(m, pu, dst, N)

        u1, statsU = _node_u1(h, msum, p['upd_W1'][:D], p['upd_W1'][D:], p['upd_b1'])
        scu, shu = _bn_coeffs(statsU, N, p['upd_g1'], p['upd_be1'])

        v, statsV = _node_u2(u1, scu, shu, p['upd_W2'], p['upd_b2'])
        scv, shv = _bn_coeffs(statsV, N, p['upd_g2'], p['upd_be2'])

        if l + 1 < 2:
            Wn = L[l + 1]['msg_W1']
            W1a_next, W1b_next = Wn[:D], Wn[D:2 * D]
        else:
            W1a_next = jnp.zeros((D, D), jnp.float32)
            W1b_next = jnp.zeros((D, D), jnp.float32)
        h, posq, A, B = _node_u3(h, v, scv, shv, psum, deg, posq,
                                 W1a_next, W1b_next)

    return _pred(h, W_pred, b_pred)


# speedup vs baseline: 1.7072x; 1.7072x over previous
import functools
import math

import jax
import jax.numpy as jnp
from jax import lax
from jax.experimental import pallas as pl
from jax.experimental.pallas import tpu as pltpu
from jax.experimental.pallas import tpu_sc as plsc

_NC = 2
_NS = 16
_SUB = 128
_RND = 512


def _blk(total, cap):
    b = min(total, cap)
    while b > 8:
        if total % b == 0 and b % 8 == 0:
            return b
        b -= 8
    return total


def _node0_body(x_ref, Win_ref, bin_ref, W1a_ref, W1b_ref, h_ref, A_ref, B_ref):
    h = jnp.dot(x_ref[...], Win_ref[...], preferred_element_type=jnp.float32, precision=jax.lax.Precision.HIGHEST) + bin_ref[...]
    h_ref[...] = h
    A_ref[...] = jnp.dot(h, W1a_ref[...], preferred_element_type=jnp.float32, precision=jax.lax.Precision.HIGHEST)
    B_ref[...] = jnp.dot(h, W1b_ref[...], preferred_element_type=jnp.float32, precision=jax.lax.Precision.HIGHEST)


def _node0(x, W_in, b_in, W1a, W1b):
    N, IN = x.shape
    D = W_in.shape[1]
    bN = _blk(N, 2048)
    grid = (N // bN,)
    out = pl.pallas_call(
        _node0_body,
        grid=grid,
        in_specs=[
            pl.BlockSpec((bN, IN), lambda i: (i, 0)),
            pl.BlockSpec((IN, D), lambda i: (0, 0)),
            pl.BlockSpec((1, D), lambda i: (0, 0)),
            pl.BlockSpec((D, D), lambda i: (0, 0)),
            pl.BlockSpec((D, D), lambda i: (0, 0)),
        ],
        out_specs=[
            pl.BlockSpec((bN, D), lambda i: (i, 0)),
            pl.BlockSpec((bN, D), lambda i: (i, 0)),
            pl.BlockSpec((bN, D), lambda i: (i, 0)),
        ],
        out_shape=[
            jax.ShapeDtypeStruct((N, D), jnp.float32),
            jax.ShapeDtypeStruct((N, D), jnp.float32),
            jax.ShapeDtypeStruct((N, D), jnp.float32),
        ],
    )(x, W_in, b_in.reshape(1, -1), W1a, W1b)
    return out


def _accum_stats(stats_ref, z, i, bE, E):
    rows = lax.broadcasted_iota(jnp.int32, (z.shape[0], 1), 0) + i * bE
    msk = (rows < E).astype(jnp.float32)
    zm = z * msk
    s = jnp.sum(zm, axis=0, keepdims=True)
    ss = jnp.sum(zm * z, axis=0, keepdims=True)
    blk = jnp.concatenate([s, ss], axis=0)

    @pl.when(i == 0)
    def _():
        stats_ref[...] = blk

    @pl.when(i > 0)
    def _():
        stats_ref[...] += blk


def _z1_block(gA_ref, gB_ref, pD_ref, pS_ref, ea_ref, W1e_ref, w1r_ref, b1_ref):
    pd = pD_ref[...][:, :4] - pS_ref[...][:, :4]
    rad = jnp.sum(pd * pd, axis=1, keepdims=True)
    return (gA_ref[...] + gB_ref[...] + rad * w1r_ref[...]
            + jnp.dot(ea_ref[...], W1e_ref[...], preferred_element_type=jnp.float32, precision=jax.lax.Precision.HIGHEST)
            + b1_ref[...])


def _p1_body(E, gA_ref, gB_ref, pD_ref, pS_ref, ea_ref, W1e_ref, w1r_ref, b1_ref, stats_ref):
    i = pl.program_id(0)
    z1 = _z1_block(gA_ref, gB_ref, pD_ref, pS_ref, ea_ref, W1e_ref, w1r_ref, b1_ref)
    _accum_stats(stats_ref, z1, i, gA_ref.shape[0], E)


def _edge_p1(E, gA, gB, pD, pS, ea, W1e, w1r, b1):
    E2, D = gA.shape
    bE = _blk(E2, 4096)
    return pl.pallas_call(
        functools.partial(_p1_body, E),
        grid=(E2 // bE,),
        in_specs=[
            pl.BlockSpec((bE, D), lambda i: (i, 0)),
            pl.BlockSpec((bE, D), lambda i: (i, 0)),
            pl.BlockSpec((bE, 16), lambda i: (i, 0)),
            pl.BlockSpec((bE, 16), lambda i: (i, 0)),
            pl.BlockSpec((bE, 4), lambda i: (i, 0)),
            pl.BlockSpec((4, D), lambda i: (0, 0)),
            pl.BlockSpec((1, D), lambda i: (0, 0)),
            pl.BlockSpec((1, D), lambda i: (0, 0)),
        ],
        out_specs=pl.BlockSpec((2, D), lambda i: (0, 0)),
        out_shape=jax.ShapeDtypeStruct((2, D), jnp.float32),
    )(gA, gB, pD, pS, ea, W1e, w1r.reshape(1, -1), b1.reshape(1, -1))


def _p2_body(E, gA_ref, gB_ref, pD_ref, pS_ref, ea_ref, W1e_ref, w1r_ref, b1_ref,
             sc_ref, sh_ref, W2_ref, b2_ref, z2_ref, stats_ref):
    i = pl.program_id(0)
    z1 = _z1_block(gA_ref, gB_ref, pD_ref, pS_ref, ea_ref, W1e_ref, w1r_ref, b1_ref)
    m1 = jnp.maximum(z1 * sc_ref[...] + sh_ref[...], 0.0)
    z2 = jnp.dot(m1, W2_ref[...], preferred_element_type=jnp.float32, precision=jax.lax.Precision.HIGHEST) + b2_ref[...]
    z2_ref[...] = z2
    _accum_stats(stats_ref, z2, i, gA_ref.shape[0], E)


def _edge_p2(E, gA, gB, pD, pS, ea, W1e, w1r, b1, sc1, sh1, W2, b2):
    E2, D = gA.shape
    bE = _blk(E2, 4096)
    return pl.pallas_call(
        functools.partial(_p2_body, E),
        grid=(E2 // bE,),
        in_specs=[
            pl.BlockSpec((bE, D), lambda i: (i, 0)),
            pl.BlockSpec((bE, D), lambda i: (i, 0)),
            pl.BlockSpec((bE, 16), lambda i: (i, 0)),
            pl.BlockSpec((bE, 16), lambda i: (i, 0)),
            pl.BlockSpec((bE, 4), lambda i: (i, 0)),
            pl.BlockSpec((4, D), lambda i: (0, 0)),
            pl.BlockSpec((1, D), lambda i: (0, 0)),
            pl.BlockSpec((1, D), lambda i: (0, 0)),
            pl.BlockSpec((1, D), lambda i: (0, 0)),
            pl.BlockSpec((1, D), lambda i: (0, 0)),
            pl.BlockSpec((D, D), lambda i: (0, 0)),
            pl.BlockSpec((1, D), lambda i: (0, 0)),
        ],
        out_specs=[
            pl.BlockSpec((bE, D), lambda i: (i, 0)),
            pl.BlockSpec((2, D), lambda i: (0, 0)),
        ],
        out_shape=[
            jax.ShapeDtypeStruct((E2, D), jnp.float32),
            jax.ShapeDtypeStruct((2, D), jnp.float32),
        ],
    )(gA, gB, pD, pS, ea, W1e, w1r.reshape(1, -1), b1.reshape(1, -1),
      sc1.reshape(1, -1), sh1.reshape(1, -1), W2, b2.reshape(1, -1))


def _p3_body(E, z2_ref, sc_ref, sh_ref, W3_ref, b3_ref, mlo_ref, mhi_ref, stats_ref):
    i = pl.program_id(0)
    m = jnp.maximum(z2_ref[...] * sc_ref[...] + sh_ref[...], 0.0)
    DH = mlo_ref.shape[1]
    mlo_ref[...] = m[:, :DH]
    mhi_ref[...] = m[:, DH:]
    z3 = jnp.dot(m, W3_ref[...], preferred_element_type=jnp.float32, precision=jax.lax.Precision.HIGHEST) + b3_ref[...]
    _accum_stats(stats_ref, z3, i, z2_ref.shape[0], E)


def _edge_p3(E, z2, sc2, sh2, W3, b3):
    E2, D = z2.shape
    DH = D // 2
    bE = _blk(E2, 4096)
    return pl.pallas_call(
        functools.partial(_p3_body, E),
        grid=(E2 // bE,),
        in_specs=[
            pl.BlockSpec((bE, D), lambda i: (i, 0)),
            pl.BlockSpec((1, D), lambda i: (0, 0)),
            pl.BlockSpec((1, D), lambda i: (0, 0)),
            pl.BlockSpec((D, D), lambda i: (0, 0)),
            pl.BlockSpec((1, D), lambda i: (0, 0)),
        ],
        out_specs=[
            pl.BlockSpec((bE, DH), lambda i: (i, 0)),
            pl.BlockSpec((bE, DH), lambda i: (i, 0)),
            pl.BlockSpec((2, D), lambda i: (0, 0)),
        ],
        out_shape=[
            jax.ShapeDtypeStruct((E2, DH), jnp.float32),
            jax.ShapeDtypeStruct((E2, DH), jnp.float32),
            jax.ShapeDtypeStruct((2, D), jnp.float32),
        ],
    )(z2, sc2.reshape(1, -1), sh2.reshape(1, -1), W3, b3.reshape(1, -1))


def _p4_body(mlo_ref, mhi_ref, pD_ref, pS_ref, sc_ref, sh_ref, W3_ref, b3_ref, W4_ref, b4_ref, pu_ref):
    m = jnp.concatenate([mlo_ref[...], mhi_ref[...]], axis=1)
    z3 = jnp.dot(m, W3_ref[...], preferred_element_type=jnp.float32, precision=jax.lax.Precision.HIGHEST) + b3_ref[...]
    pm1 = jnp.maximum(z3 * sc_ref[...] + sh_ref[...], 0.0)
    pm = jnp.dot(pm1, W4_ref[...], preferred_element_type=jnp.float32, precision=jax.lax.Precision.HIGHEST) + b4_ref[...]
    pu = (pD_ref[...][:, :4] - pS_ref[...][:, :4]) * pm
    lane = lax.broadcasted_iota(jnp.int32, pu.shape, 1)
    pu_ref[...] = jnp.where(lane == 3, 1.0, pu)


def _edge_p4(mlo, mhi, pD, pS, sc3, sh3, W3, b3, W4, b4):
    E2, DH = mlo.shape
    D = 2 * DH
    bE = _blk(E2, 4096)
    return pl.pallas_call(
        _p4_body,
        grid=(E2 // bE,),
        in_specs=[
            pl.BlockSpec((bE, DH), lambda i: (i, 0)),
            pl.BlockSpec((bE, DH), lambda i: (i, 0)),
            pl.BlockSpec((bE, 16), lambda i: (i, 0)),
            pl.BlockSpec((bE, 16), lambda i: (i, 0)),
            pl.BlockSpec((1, D), lambda i: (0, 0)),
            pl.BlockSpec((1, D), lambda i: (0, 0)),
            pl.BlockSpec((D, D), lambda i: (0, 0)),
            pl.BlockSpec((1, D), lambda i: (0, 0)),
            pl.BlockSpec((D, 1), lambda i: (0, 0)),
            pl.BlockSpec((1, 1), lambda i: (0, 0)),
        ],
        out_specs=pl.BlockSpec((bE, 4), lambda i: (i, 0)),
        out_shape=jax.ShapeDtypeStruct((E2, 4), jnp.float32),
    )(mlo, mhi, pD, pS, sc3.reshape(1, -1), sh3.reshape(1, -1), W3, b3.reshape(1, -1),
      W4, b4.reshape(1, 1))


def _u1_body(h_ref, mlo_ref, mhi_ref, Wa_ref, Wblo_ref, Wbhi_ref, b_ref, u1_ref, stats_ref):
    i = pl.program_id(0)
    u1 = (jnp.dot(h_ref[...], Wa_ref[...], preferred_element_type=jnp.float32, precision=jax.lax.Precision.HIGHEST)
          + jnp.dot(mlo_ref[...], Wblo_ref[...], preferred_element_type=jnp.float32, precision=jax.lax.Precision.HIGHEST)
          + jnp.dot(mhi_ref[...], Wbhi_ref[...], preferred_element_type=jnp.float32, precision=jax.lax.Precision.HIGHEST)
          + b_ref[...])
    u1_ref[...] = u1
    _accum_stats(stats_ref, u1, i, u1.shape[0], u1.shape[0] * pl.num_programs(0))


def _node_u1(h, mslo, mshi, Wa, Wblo, Wbhi, b):
    N, D = h.shape
    DH = mslo.shape[1]
    bN = _blk(N, 2048)
    return pl.pallas_call(
        _u1_body,
        grid=(N // bN,),
        in_specs=[
            pl.BlockSpec((bN, D), lambda i: (i, 0)),
            pl.BlockSpec((bN, DH), lambda i: (i, 0)),
            pl.BlockSpec((bN, DH), lambda i: (i, 0)),
            pl.BlockSpec((D, D), lambda i: (0, 0)),
            pl.BlockSpec((DH, D), lambda i: (0, 0)),
            pl.BlockSpec((DH, D), lambda i: (0, 0)),
            pl.BlockSpec((1, D), lambda i: (0, 0)),
        ],
        out_specs=[
            pl.BlockSpec((bN, D), lambda i: (i, 0)),
            pl.BlockSpec((2, D), lambda i: (0, 0)),
        ],
        out_shape=[
            jax.ShapeDtypeStruct((N, D), jnp.float32),
            jax.ShapeDtypeStruct((2, D), jnp.float32),
        ],
    )(h, mslo, mshi, Wa, Wblo, Wbhi, b.reshape(1, -1))


def _u2_body(u1_ref, sc_ref, sh_ref, W_ref, b_ref, v_ref, stats_ref):
    i = pl.program_id(0)
    r = jnp.maximum(u1_ref[...] * sc_ref[...] + sh_ref[...], 0.0)
    v = jnp.dot(r, W_ref[...], preferred_element_type=jnp.float32, precision=jax.lax.Precision.HIGHEST) + b_ref[...]
    v_ref[...] = v
    _accum_stats(stats_ref, v, i, v.shape[0], v.shape[0] * pl.num_programs(0))


def _node_u2(u1, scu, shu, W, b):
    N, D = u1.shape
    bN = _blk(N, 2048)
    return pl.pallas_call(
        _u2_body,
        grid=(N // bN,),
        in_specs=[
            pl.BlockSpec((bN, D), lambda i: (i, 0)),
            pl.BlockSpec((1, D), lambda i: (0, 0)),
            pl.BlockSpec((1, D), lambda i: (0, 0)),
            pl.BlockSpec((D, D), lambda i: (0, 0)),
            pl.BlockSpec((1, D), lambda i: (0, 0)),
        ],
        out_specs=[
            pl.BlockSpec((bN, D), lambda i: (i, 0)),
            pl.BlockSpec((2, D), lambda i: (0, 0)),
        ],
        out_shape=[
            jax.ShapeDtypeStruct((N, D), jnp.float32),
            jax.ShapeDtypeStruct((2, D), jnp.float32),
        ],
    )(u1, scu.reshape(1, -1), shu.reshape(1, -1), W, b.reshape(1, -1))


def _u3_body(h_ref, v_ref, sc_ref, sh_ref, ps0_ref, ps1_ref, posq_ref,
             W1a_ref, W1b_ref, h2_ref, pq2_ref, A_ref, B_ref):
    h2 = h_ref[...] + jnp.maximum(v_ref[...] * sc_ref[...] + sh_ref[...], 0.0)
    h2_ref[...] = h2
    psum = ps0_ref[...] + ps1_ref[...]
    deg = psum[:, 3:4]
    pq2 = posq_ref[...][:, :4] + psum / jnp.maximum(deg, 1.0)
    pq2_ref[...] = jnp.pad(pq2, ((0, 0), (0, 12)))
    A_ref[...] = jnp.dot(h2, W1a_ref[...], preferred_element_type=jnp.float32, precision=jax.lax.Precision.HIGHEST)
    B_ref[...] = jnp.dot(h2, W1b_ref[...], preferred_element_type=jnp.float32, precision=jax.lax.Precision.HIGHEST)


def _node_u3(h, v, scv, shv, ps0, ps1, posq, W1a_next, W1b_next):
    N, D = h.shape
    bN = _blk(N, 2048)
    return pl.pallas_call(
        _u3_body,
        grid=(N // bN,),
        in_specs=[
            pl.BlockSpec((bN, D), lambda i: (i, 0)),
            pl.BlockSpec((bN, D), lambda i: (i, 0)),
            pl.BlockSpec((1, D), lambda i: (0, 0)),
            pl.BlockSpec((1, D), lambda i: (0, 0)),
            pl.BlockSpec((bN, 4), lambda i: (i, 0)),
            pl.BlockSpec((bN, 4), lambda i: (i, 0)),
            pl.BlockSpec((bN, 16), lambda i: (i, 0)),
            pl.BlockSpec((D, D), lambda i: (0, 0)),
            pl.BlockSpec((D, D), lambda i: (0, 0)),
        ],
        out_specs=[
            pl.BlockSpec((bN, D), lambda i: (i, 0)),
            pl.BlockSpec((bN, 16), lambda i: (i, 0)),
            pl.BlockSpec((bN, D), lambda i: (i, 0)),
            pl.BlockSpec((bN, D), lambda i: (i, 0)),
        ],
        out_shape=[
            jax.ShapeDtypeStruct((N, D), jnp.float32),
            jax.ShapeDtypeStruct((N, 16), jnp.float32),
            jax.ShapeDtypeStruct((N, D), jnp.float32),
            jax.ShapeDtypeStruct((N, D), jnp.float32),
        ],
    )(h, v, scv.reshape(1, -1), shv.reshape(1, -1), ps0, ps1, posq,
      W1a_next, W1b_next)


def _pred_body(h_ref, wp_ref, bp_ref, out_ref, acc_ref):
    i = pl.program_id(0)

    @pl.when(i == 0)
    def _():
        acc_ref[...] = jnp.zeros_like(acc_ref)

    acc_ref[...] += jnp.sum(h_ref[...], axis=0, keepdims=True)

    @pl.when(i == pl.num_programs(0) - 1)
    def _():
        n = pl.num_programs(0) * h_ref.shape[0]
        pooled = acc_ref[...] / jnp.float32(n)
        out_ref[...] = jnp.dot(pooled, wp_ref[...], preferred_element_type=jnp.float32, precision=jax.lax.Precision.HIGHEST) + bp_ref[...]


def _pred(h, W_pred, b_pred):
    N, D = h.shape
    bN = _blk(N, 2048)
    out = pl.pallas_call(
        _pred_body,
        grid=(N // bN,),
        in_specs=[
            pl.BlockSpec((bN, D), lambda i: (i, 0)),
            pl.BlockSpec((D, 1), lambda i: (0, 0)),
            pl.BlockSpec((1, 1), lambda i: (0, 0)),
        ],
        out_specs=pl.BlockSpec((1, 1), lambda i: (0, 0)),
        out_shape=jax.ShapeDtypeStruct((1, 1), jnp.float32),
        scratch_shapes=[pltpu.VMEM((1, D), jnp.float32)],
    )(h, W_pred, b_pred.reshape(1, 1))
    return out.reshape(-1)


def _sc_gather(A, B, posq, dst3d, src3d, E2):
    N, D = A.shape
    EPW = E2 // (_NC * _NS)
    R = EPW // 1024
    mesh = plsc.VectorSubcoreMesh(core_axis_name="c", subcore_axis_name="s")

    @functools.partial(
        pl.kernel, mesh=mesh,
        compiler_params=pltpu.CompilerParams(use_tc_tiling_on_sc=False),
        out_type=[
            jax.ShapeDtypeStruct((E2, D), jnp.float32),
            jax.ShapeDtypeStruct((E2, D), jnp.float32),
            jax.ShapeDtypeStruct((E2, 16), jnp.float32),
            jax.ShapeDtypeStruct((E2, 16), jnp.float32),
        ],
        scratch_types=[
            pltpu.VMEM((8, _SUB), jnp.int32),
            pltpu.VMEM((8, _SUB), jnp.int32),
            pltpu.VMEM((_RND, D), jnp.float32),
            pltpu.VMEM((_RND, D), jnp.float32),
            pltpu.VMEM((_RND, 16), jnp.float32),
            pltpu.VMEM((_RND, 16), jnp.float32),
            pltpu.SemaphoreType.DMA,
        ],
    )
    def k(A_h, B_h, pq_h, dst_h, src_h, gA_h, gB_h, pD_h, pS_h,
          di, si, bufA, bufB, bufD, bufS, sem):
        c = lax.axis_index("c")
        s = lax.axis_index("s")
        w = s * _NC + c
        base0 = w * EPW

        def round_(r, carry):
            blk = w * R + r
            pltpu.sync_copy(dst_h.at[blk], di)
            pltpu.sync_copy(src_h.at[blk], si)
            for h in range(2):
                base = base0 + r * 1024 + h * _RND
                cps = []
                for g in range(4):
                    row = h * 4 + g
                    sl = pl.ds(g * _SUB, _SUB)
                    cps.append(pltpu.async_copy(A_h.at[di.at[row]], bufA.at[sl, :], sem))
                    cps.append(pltpu.async_copy(B_h.at[si.at[row]], bufB.at[sl, :], sem))
                    cps.append(pltpu.async_copy(pq_h.at[di.at[row]], bufD.at[sl, :], sem))
                    cps.append(pltpu.async_copy(pq_h.at[si.at[row]], bufS.at[sl, :], sem))
                for cp in cps:
                    cp.wait()
                pltpu.sync_copy(bufA, gA_h.at[pl.ds(base, _RND), :])
                pltpu.sync_copy(bufB, gB_h.at[pl.ds(base, _RND), :])
                pltpu.sync_copy(bufD, pD_h.at[pl.ds(base, _RND), :])
                pltpu.sync_copy(bufS, pS_h.at[pl.ds(base, _RND), :])
            return carry

        lax.fori_loop(0, R, round_, 0)

    return k(A, B, posq, dst3d, src3d)


def _sc_scatter_m(mlo, mhi, dst3d, z32, NA):
    E2 = mlo.shape[0]
    DH = mlo.shape[1]
    EPT = E2 // _NS
    R = EPT // 1024
    NPT = NA // _NS
    mesh = plsc.VectorSubcoreMesh(core_axis_name="c", subcore_axis_name="s")

    @functools.partial(
        pl.kernel, mesh=mesh,
        compiler_params=pltpu.CompilerParams(use_tc_tiling_on_sc=False),
        out_type=[
            jax.ShapeDtypeStruct((NA, DH), jnp.float32),
            jax.ShapeDtypeStruct((NA, DH), jnp.float32),
        ],
        scratch_types=[
            pltpu.VMEM((8, _SUB), jnp.int32),
            pltpu.VMEM((_RND, 32), jnp.float32),
            pltpu.VMEM_SHARED((NA, 32), jnp.float32),
            pltpu.SemaphoreType.DMA,
        ],
    )
    def k(mlo_h, mhi_h, dst_h, z32_h, mslo_h, mshi_h, di, mbuf, acc_m, sem):
        c = lax.axis_index("c")
        s = lax.axis_index("s")
        nsl = pl.ds(s * NPT, NPT)
        pltpu.sync_copy(z32_h.at[nsl, :], acc_m.at[nsl, :])
        plsc.subcore_barrier()

        def round_(r, carry):
            blk = s * R + r
            pltpu.sync_copy(dst_h.at[blk], di)
            for h in range(2):
                esl = pl.ds(s * EPT + r * 1024 + h * _RND, _RND)

                @pl.when(c == 0)
                def _():
                    pltpu.sync_copy(mlo_h.at[esl, :], mbuf)

                @pl.when(c == 1)
                def _():
                    pltpu.sync_copy(mhi_h.at[esl, :], mbuf)

                for g in range(4):
                    row = h * 4 + g
                    sl = pl.ds(g * _SUB, _SUB)
                    pltpu.sync_copy(mbuf.at[sl, :], acc_m.at[di.at[row]], add=True)

            return carry

        lax.fori_loop(0, R, round_, 0)
        plsc.subcore_barrier()

        @pl.when(c == 0)
        def _():
            pltpu.sync_copy(acc_m.at[nsl, :], mslo_h.at[nsl, :])

        @pl.when(c == 1)
        def _():
            pltpu.sync_copy(acc_m.at[nsl, :], mshi_h.at[nsl, :])

    return k(mlo, mhi, dst3d, z32)


def _sc_scatter_p(pu, dst3d, z4, NA):
    E2 = pu.shape[0]
    EPT = E2 // (_NC * _NS)
    R = EPT // 1024
    NPT = NA // _NS
    mesh = plsc.VectorSubcoreMesh(core_axis_name="c", subcore_axis_name="s")

    @functools.partial(
        pl.kernel, mesh=mesh,
        compiler_params=pltpu.CompilerParams(use_tc_tiling_on_sc=False),
        out_type=[
            jax.ShapeDtypeStruct((NA, 4), jnp.float32),
            jax.ShapeDtypeStruct((NA, 4), jnp.float32),
        ],
        scratch_types=[
            pltpu.VMEM((8, _SUB), jnp.int32),
            pltpu.VMEM((_RND, 4), jnp.float32),
            pltpu.VMEM_SHARED((NA, 4), jnp.float32),
            pltpu.SemaphoreType.DMA,
        ],
    )
    def k(pu_h, dst_h, z4_h, ps0_h, ps1_h, di, pbuf, acc_p, sem):
        c = lax.axis_index("c")
        s = lax.axis_index("s")
        w = c * _NS + s
        nsl = pl.ds(s * NPT, NPT)
        pltpu.sync_copy(z4_h.at[nsl, :], acc_p.at[nsl, :])
        plsc.subcore_barrier()

        def round_(r, carry):
            blk = w * R + r
            pltpu.sync_copy(dst_h.at[blk], di)
            for h in range(2):
                esl = pl.ds(w * EPT + r * 1024 + h * _RND, _RND)
                pltpu.sync_copy(pu_h.at[esl, :], pbuf)
                for g in range(4):
                    row = h * 4 + g
                    sl = pl.ds(g * _SUB, _SUB)
                    pltpu.sync_copy(pbuf.at[sl, :], acc_p.at[di.at[row]], add=True)

            return carry

        lax.fori_loop(0, R, round_, 0)
        plsc.subcore_barrier()

        @pl.when(c == 0)
        def _():
            pltpu.sync_copy(acc_p.at[nsl, :], ps0_h.at[nsl, :])

        @pl.when(c == 1)
        def _():
            pltpu.sync_copy(acc_p.at[nsl, :], ps1_h.at[nsl, :])

    return k(pu, dst3d, z4)


def _bn_coeffs(stats, count, gamma, beta):
    s, ss = stats[0], stats[1]
    mean = s / count
    var = ss / count - mean * mean
    sc = gamma / jnp.sqrt(var + 1e-5)
    sh = beta - mean * sc
    return sc, sh


def kernel(W_in, b_in, l0_msg_W1, l0_msg_b1, l0_msg_g1, l0_msg_be1, l0_msg_W2, l0_msg_b2, l0_msg_g2, l0_msg_be2, l0_pos_W1, l0_pos_b1, l0_pos_g1, l0_pos_be1, l0_pos_W2, l0_pos_b2, l0_upd_W1, l0_upd_b1, l0_upd_g1, l0_upd_be1, l0_upd_W2, l0_upd_b2, l0_upd_g2, l0_upd_be2, l1_msg_W1, l1_msg_b1, l1_msg_g1, l1_msg_be1, l1_msg_W2, l1_msg_b2, l1_msg_g2, l1_msg_be2, l1_pos_W1, l1_pos_b1, l1_pos_g1, l1_pos_be1, l1_pos_W2, l1_pos_b2, l1_upd_W1, l1_upd_b1, l1_upd_g1, l1_upd_be1, l1_upd_W2, l1_upd_b2, l1_upd_g2, l1_upd_be2, W_pred, b_pred, x, pos, edge_index, edge_attr, batch):
    N = x.shape[0]
    E = edge_index.shape[1]
    D = W_in.shape[1]
    grain = _NC * _NS * 1024
    E2 = ((E + grain - 1) // grain) * grain
    NA = ((N + 1 + 127) // 128) * 128
    src = jnp.concatenate([edge_index[0], jnp.zeros((E2 - E,), jnp.int32)])
    dst = jnp.concatenate([edge_index[1], jnp.full((E2 - E,), N, jnp.int32)])
    src3d = src.reshape(E2 // 1024, 8, _SUB)
    dst3d = dst.reshape(E2 // 1024, 8, _SUB)
    eap = jnp.concatenate([edge_attr, jnp.zeros((E2 - E, 4), jnp.float32)])
    z32 = jnp.zeros((NA, 32), jnp.float32)
    z4 = jnp.zeros((NA, 4), jnp.float32)
    posq = jnp.pad(pos, ((0, 0), (0, 13)))

    L = [
        dict(msg_W1=l0_msg_W1, msg_b1=l0_msg_b1, msg_g1=l0_msg_g1, msg_be1=l0_msg_be1,
             msg_W2=l0_msg_W2, msg_b2=l0_msg_b2, msg_g2=l0_msg_g2, msg_be2=l0_msg_be2,
             pos_W1=l0_pos_W1, pos_b1=l0_pos_b1, pos_g1=l0_pos_g1, pos_be1=l0_pos_be1,
             pos_W2=l0_pos_W2, pos_b2=l0_pos_b2,
             upd_W1=l0_upd_W1, upd_b1=l0_upd_b1, upd_g1=l0_upd_g1, upd_be1=l0_upd_be1,
             upd_W2=l0_upd_W2, upd_b2=l0_upd_b2, upd_g2=l0_upd_g2, upd_be2=l0_upd_be2),
        dict(msg_W1=l1_msg_W1, msg_b1=l1_msg_b1, msg_g1=l1_msg_g1, msg_be1=l1_msg_be1,
             msg_W2=l1_msg_W2, msg_b2=l1_msg_b2, msg_g2=l1_msg_g2, msg_be2=l1_msg_be2,
             pos_W1=l1_pos_W1, pos_b1=l1_pos_b1, pos_g1=l1_pos_g1, pos_be1=l1_pos_be1,
             pos_W2=l1_pos_W2, pos_b2=l1_pos_b2,
             upd_W1=l1_upd_W1, upd_b1=l1_upd_b1, upd_g1=l1_upd_g1, upd_be1=l1_upd_be1,
             upd_W2=l1_upd_W2, upd_b2=l1_upd_b2, upd_g2=l1_upd_g2, upd_be2=l1_upd_be2),
    ]

    W1 = L[0]['msg_W1']
    h, A, B = _node0(x, W_in, b_in, W1[:D], W1[D:2 * D])

    for l in range(2):
        p = L[l]
        W1 = p['msg_W1']
        w1r = W1[2 * D]
        W1e = W1[2 * D + 1:]

        gA, gB, pD, pS = _sc_gather(A, B, posq, dst3d, src3d, E2)

        stats1 = _edge_p1(E, gA, gB, pD, pS, eap, W1e, w1r, p['msg_b1'])
        sc1, sh1 = _bn_coeffs(stats1, E, p['msg_g1'], p['msg_be1'])

        z2, stats2 = _edge_p2(E, gA, gB, pD, pS, eap, W1e, w1r, p['msg_b1'],
                              sc1, sh1, p['msg_W2'], p['msg_b2'])
        sc2, sh2 = _bn_coeffs(stats2, E, p['msg_g2'], p['msg_be2'])

        mlo, mhi, stats3 = _edge_p3(E, z2, sc2, sh2, p['pos_W1'], p['pos_b1'])
        sc3, sh3 = _bn_coeffs(stats3, E, p['pos_g1'], p['pos_be1'])

        pu = _edge_p4(mlo, mhi, pD, pS, sc3, sh3, p['pos_W1'], p['pos_b1'],
                      p['pos_W2'], p['pos_b2'])

        mslo, mshi = _sc_scatter_m(mlo, mhi, dst3d, z32, NA)
        ps0, ps1 = _sc_scatter_p(pu, dst3d, z4, NA)

        Wu = p['upd_W1']
        u1, statsU = _node_u1(h, mslo, mshi, Wu[:D], Wu[D:D + D // 2],
                              Wu[D + D // 2:], p['upd_b1'])
        scu, shu = _bn_coeffs(statsU, N, p['upd_g1'], p['upd_be1'])

        v, statsV = _node_u2(u1, scu, shu, p['upd_W2'], p['upd_b2'])
        scv, shv = _bn_coeffs(statsV, N, p['upd_g2'], p['upd_be2'])

        if l + 1 < 2:
            Wn = L[l + 1]['msg_W1']
            W1a_next, W1b_next = Wn[:D], Wn[D:2 * D]
        else:
            W1a_next = jnp.zeros((D, D), jnp.float32)
            W1b_next = jnp.zeros((D, D), jnp.float32)
        h, posq, A, B = _node_u3(h, v, scv, shv, ps0, ps1, posq,
                                 W1a_next, W1b_next)

    return _pred(h, W_pred, b_pred)


# speedup vs baseline: 2.0510x; 1.2014x over previous
import functools
import math

import jax
import jax.numpy as jnp
from jax import lax
from jax.experimental import pallas as pl
from jax.experimental.pallas import tpu as pltpu
from jax.experimental.pallas import tpu_sc as plsc

_NC = 2
_NS = 16
_SUB = 128
_RND = 512


def _blk(total, cap):
    b = min(total, cap)
    while b > 8:
        if total % b == 0 and b % 8 == 0:
            return b
        b -= 8
    return total


def _node0_body(x_ref, Win_ref, bin_ref, W1a_ref, W1b_ref, h_ref, A_ref, B_ref):
    h = jnp.dot(x_ref[...], Win_ref[...], preferred_element_type=jnp.float32, precision=jax.lax.Precision.HIGHEST) + bin_ref[...]
    h_ref[...] = h
    A_ref[...] = jnp.dot(h, W1a_ref[...], preferred_element_type=jnp.float32, precision=jax.lax.Precision.HIGHEST)
    B_ref[...] = jnp.dot(h, W1b_ref[...], preferred_element_type=jnp.float32, precision=jax.lax.Precision.HIGHEST)


def _node0(x, W_in, b_in, W1a, W1b):
    N, IN = x.shape
    D = W_in.shape[1]
    bN = _blk(N, 2048)
    grid = (N // bN,)
    out = pl.pallas_call(
        _node0_body,
        grid=grid,
        in_specs=[
            pl.BlockSpec((bN, IN), lambda i: (i, 0)),
            pl.BlockSpec((IN, D), lambda i: (0, 0)),
            pl.BlockSpec((1, D), lambda i: (0, 0)),
            pl.BlockSpec((D, D), lambda i: (0, 0)),
            pl.BlockSpec((D, D), lambda i: (0, 0)),
        ],
        out_specs=[
            pl.BlockSpec((bN, D), lambda i: (i, 0)),
            pl.BlockSpec((bN, D), lambda i: (i, 0)),
            pl.BlockSpec((bN, D), lambda i: (i, 0)),
        ],
        out_shape=[
            jax.ShapeDtypeStruct((N, D), jnp.float32),
            jax.ShapeDtypeStruct((N, D), jnp.float32),
            jax.ShapeDtypeStruct((N, D), jnp.float32),
        ],
    )(x, W_in, b_in.reshape(1, -1), W1a, W1b)
    return out


def _accum_stats(stats_ref, z, i, bE, E):
    rows = lax.broadcasted_iota(jnp.int32, (z.shape[0], 1), 0) + i * bE
    msk = (rows < E).astype(jnp.float32)
    zm = z * msk
    s = jnp.sum(zm, axis=0, keepdims=True)
    ss = jnp.sum(zm * z, axis=0, keepdims=True)
    blk = jnp.concatenate([s, ss], axis=0)

    @pl.when(i == 0)
    def _():
        stats_ref[...] = blk

    @pl.when(i > 0)
    def _():
        stats_ref[...] += blk


def _z1_block(gA_ref, gB_ref, pD_ref, pS_ref, ea_ref, W1e_ref, w1r_ref, b1_ref):
    pd = pD_ref[...][:, :4] - pS_ref[...][:, :4]
    rad = jnp.sum(pd * pd, axis=1, keepdims=True)
    return (gA_ref[...] + gB_ref[...] + rad * w1r_ref[...]
            + jnp.dot(ea_ref[...], W1e_ref[...], preferred_element_type=jnp.float32, precision=jax.lax.Precision.HIGHEST)
            + b1_ref[...])


def _p1_body(E, gA_ref, gB_ref, pD_ref, pS_ref, ea_ref, W1e_ref, w1r_ref, b1_ref, stats_ref):
    i = pl.program_id(0)
    z1 = _z1_block(gA_ref, gB_ref, pD_ref, pS_ref, ea_ref, W1e_ref, w1r_ref, b1_ref)
    _accum_stats(stats_ref, z1, i, gA_ref.shape[0], E)


def _edge_p1(E, gA, gB, pD, pS, ea, W1e, w1r, b1):
    E2, D = gA.shape
    bE = _blk(E2, 4096)
    return pl.pallas_call(
        functools.partial(_p1_body, E),
        grid=(E2 // bE,),
        in_specs=[
            pl.BlockSpec((bE, D), lambda i: (i, 0)),
            pl.BlockSpec((bE, D), lambda i: (i, 0)),
            pl.BlockSpec((bE, 16), lambda i: (i, 0)),
            pl.BlockSpec((bE, 16), lambda i: (i, 0)),
            pl.BlockSpec((bE, 4), lambda i: (i, 0)),
            pl.BlockSpec((4, D), lambda i: (0, 0)),
            pl.BlockSpec((1, D), lambda i: (0, 0)),
            pl.BlockSpec((1, D), lambda i: (0, 0)),
        ],
        out_specs=pl.BlockSpec((2, D), lambda i: (0, 0)),
        out_shape=jax.ShapeDtypeStruct((2, D), jnp.float32),
    )(gA, gB, pD, pS, ea, W1e, w1r.reshape(1, -1), b1.reshape(1, -1))


def _p2_body(E, gA_ref, gB_ref, pD_ref, pS_ref, ea_ref, W1e_ref, w1r_ref, b1_ref,
             sc_ref, sh_ref, W2_ref, b2_ref, z2_ref, stats_ref):
    i = pl.program_id(0)
    z1 = _z1_block(gA_ref, gB_ref, pD_ref, pS_ref, ea_ref, W1e_ref, w1r_ref, b1_ref)
    m1 = jnp.maximum(z1 * sc_ref[...] + sh_ref[...], 0.0)
    z2 = jnp.dot(m1, W2_ref[...], preferred_element_type=jnp.float32, precision=jax.lax.Precision.DEFAULT) + b2_ref[...]
    z2_ref[...] = z2
    _accum_stats(stats_ref, z2, i, gA_ref.shape[0], E)


def _edge_p2(E, gA, gB, pD, pS, ea, W1e, w1r, b1, sc1, sh1, W2, b2):
    E2, D = gA.shape
    bE = _blk(E2, 4096)
    return pl.pallas_call(
        functools.partial(_p2_body, E),
        grid=(E2 // bE,),
        in_specs=[
            pl.BlockSpec((bE, D), lambda i: (i, 0)),
            pl.BlockSpec((bE, D), lambda i: (i, 0)),
            pl.BlockSpec((bE, 16), lambda i: (i, 0)),
            pl.BlockSpec((bE, 16), lambda i: (i, 0)),
            pl.BlockSpec((bE, 4), lambda i: (i, 0)),
            pl.BlockSpec((4, D), lambda i: (0, 0)),
            pl.BlockSpec((1, D), lambda i: (0, 0)),
            pl.BlockSpec((1, D), lambda i: (0, 0)),
            pl.BlockSpec((1, D), lambda i: (0, 0)),
            pl.BlockSpec((1, D), lambda i: (0, 0)),
            pl.BlockSpec((D, D), lambda i: (0, 0)),
            pl.BlockSpec((1, D), lambda i: (0, 0)),
        ],
        out_specs=[
            pl.BlockSpec((bE, D), lambda i: (i, 0)),
            pl.BlockSpec((2, D), lambda i: (0, 0)),
        ],
        out_shape=[
            jax.ShapeDtypeStruct((E2, D), jnp.float32),
            jax.ShapeDtypeStruct((2, D), jnp.float32),
        ],
    )(gA, gB, pD, pS, ea, W1e, w1r.reshape(1, -1), b1.reshape(1, -1),
      sc1.reshape(1, -1), sh1.reshape(1, -1), W2, b2.reshape(1, -1))


def _p3_body(E, z2_ref, sc_ref, sh_ref, W3_ref, b3_ref, mlo_ref, mhi_ref, stats_ref):
    i = pl.program_id(0)
    m = jnp.maximum(z2_ref[...] * sc_ref[...] + sh_ref[...], 0.0)
    DH = mlo_ref.shape[1]
    mlo_ref[...] = m[:, :DH]
    mhi_ref[...] = m[:, DH:]
    z3 = jnp.dot(m, W3_ref[...], preferred_element_type=jnp.float32, precision=jax.lax.Precision.DEFAULT) + b3_ref[...]
    _accum_stats(stats_ref, z3, i, z2_ref.shape[0], E)


def _edge_p3(E, z2, sc2, sh2, W3, b3):
    E2, D = z2.shape
    DH = D // 2
    bE = _blk(E2, 4096)
    return pl.pallas_call(
        functools.partial(_p3_body, E),
        grid=(E2 // bE,),
        in_specs=[
            pl.BlockSpec((bE, D), lambda i: (i, 0)),
            pl.BlockSpec((1, D), lambda i: (0, 0)),
            pl.BlockSpec((1, D), lambda i: (0, 0)),
            pl.BlockSpec((D, D), lambda i: (0, 0)),
            pl.BlockSpec((1, D), lambda i: (0, 0)),
        ],
        out_specs=[
            pl.BlockSpec((bE, DH), lambda i: (i, 0)),
            pl.BlockSpec((bE, DH), lambda i: (i, 0)),
            pl.BlockSpec((2, D), lambda i: (0, 0)),
        ],
        out_shape=[
            jax.ShapeDtypeStruct((E2, DH), jnp.float32),
            jax.ShapeDtypeStruct((E2, DH), jnp.float32),
            jax.ShapeDtypeStruct((2, D), jnp.float32),
        ],
    )(z2, sc2.reshape(1, -1), sh2.reshape(1, -1), W3, b3.reshape(1, -1))


def _p4_body(mlo_ref, mhi_ref, pD_ref, pS_ref, sc_ref, sh_ref, W3_ref, b3_ref, W4_ref, b4_ref, pu_ref):
    m = jnp.concatenate([mlo_ref[...], mhi_ref[...]], axis=1)
    z3 = jnp.dot(m, W3_ref[...], preferred_element_type=jnp.float32, precision=jax.lax.Precision.DEFAULT) + b3_ref[...]
    pm1 = jnp.maximum(z3 * sc_ref[...] + sh_ref[...], 0.0)
    pm = jnp.dot(pm1, W4_ref[...], preferred_element_type=jnp.float32, precision=jax.lax.Precision.DEFAULT) + b4_ref[...]
    pu = (pD_ref[...][:, :4] - pS_ref[...][:, :4]) * pm
    lane = lax.broadcasted_iota(jnp.int32, pu.shape, 1)
    pu_ref[...] = jnp.where(lane == 3, 1.0, pu)


def _edge_p4(mlo, mhi, pD, pS, sc3, sh3, W3, b3, W4, b4):
    E2, DH = mlo.shape
    D = 2 * DH
    bE = _blk(E2, 4096)
    return pl.pallas_call(
        _p4_body,
        grid=(E2 // bE,),
        in_specs=[
            pl.BlockSpec((bE, DH), lambda i: (i, 0)),
            pl.BlockSpec((bE, DH), lambda i: (i, 0)),
            pl.BlockSpec((bE, 16), lambda i: (i, 0)),
            pl.BlockSpec((bE, 16), lambda i: (i, 0)),
            pl.BlockSpec((1, D), lambda i: (0, 0)),
            pl.BlockSpec((1, D), lambda i: (0, 0)),
            pl.BlockSpec((D, D), lambda i: (0, 0)),
            pl.BlockSpec((1, D), lambda i: (0, 0)),
            pl.BlockSpec((D, 1), lambda i: (0, 0)),
            pl.BlockSpec((1, 1), lambda i: (0, 0)),
        ],
        out_specs=pl.BlockSpec((bE, 4), lambda i: (i, 0)),
        out_shape=jax.ShapeDtypeStruct((E2, 4), jnp.float32),
    )(mlo, mhi, pD, pS, sc3.reshape(1, -1), sh3.reshape(1, -1), W3, b3.reshape(1, -1),
      W4, b4.reshape(1, 1))


def _u1_body(h_ref, mlo_ref, mhi_ref, Wa_ref, Wblo_ref, Wbhi_ref, b_ref, u1_ref, stats_ref):
    i = pl.program_id(0)
    u1 = (jnp.dot(h_ref[...], Wa_ref[...], preferred_element_type=jnp.float32, precision=jax.lax.Precision.HIGHEST)
          + jnp.dot(mlo_ref[...], Wblo_ref[...], preferred_element_type=jnp.float32, precision=jax.lax.Precision.HIGHEST)
          + jnp.dot(mhi_ref[...], Wbhi_ref[...], preferred_element_type=jnp.float32, precision=jax.lax.Precision.HIGHEST)
          + b_ref[...])
    u1_ref[...] = u1
    _accum_stats(stats_ref, u1, i, u1.shape[0], u1.shape[0] * pl.num_programs(0))


def _node_u1(h, mslo, mshi, Wa, Wblo, Wbhi, b):
    N, D = h.shape
    DH = mslo.shape[1]
    bN = _blk(N, 2048)
    return pl.pallas_call(
        _u1_body,
        grid=(N // bN,),
        in_specs=[
            pl.BlockSpec((bN, D), lambda i: (i, 0)),
            pl.BlockSpec((bN, DH), lambda i: (i, 0)),
            pl.BlockSpec((bN, DH), lambda i: (i, 0)),
            pl.BlockSpec((D, D), lambda i: (0, 0)),
            pl.BlockSpec((DH, D), lambda i: (0, 0)),
            pl.BlockSpec((DH, D), lambda i: (0, 0)),
            pl.BlockSpec((1, D), lambda i: (0, 0)),
        ],
        out_specs=[
            pl.BlockSpec((bN, D), lambda i: (i, 0)),
            pl.BlockSpec((2, D), lambda i: (0, 0)),
        ],
        out_shape=[
            jax.ShapeDtypeStruct((N, D), jnp.float32),
            jax.ShapeDtypeStruct((2, D), jnp.float32),
        ],
    )(h, mslo, mshi, Wa, Wblo, Wbhi, b.reshape(1, -1))


def _u2_body(u1_ref, sc_ref, sh_ref, W_ref, b_ref, v_ref, stats_ref):
    i = pl.program_id(0)
    r = jnp.maximum(u1_ref[...] * sc_ref[...] + sh_ref[...], 0.0)
    v = jnp.dot(r, W_ref[...], preferred_element_type=jnp.float32, precision=jax.lax.Precision.HIGHEST) + b_ref[...]
    v_ref[...] = v
    _accum_stats(stats_ref, v, i, v.shape[0], v.shape[0] * pl.num_programs(0))


def _node_u2(u1, scu, shu, W, b):
    N, D = u1.shape
    bN = _blk(N, 2048)
    return pl.pallas_call(
        _u2_body,
        grid=(N // bN,),
        in_specs=[
            pl.BlockSpec((bN, D), lambda i: (i, 0)),
            pl.BlockSpec((1, D), lambda i: (0, 0)),
            pl.BlockSpec((1, D), lambda i: (0, 0)),
            pl.BlockSpec((D, D), lambda i: (0, 0)),
            pl.BlockSpec((1, D), lambda i: (0, 0)),
        ],
        out_specs=[
            pl.BlockSpec((bN, D), lambda i: (i, 0)),
            pl.BlockSpec((2, D), lambda i: (0, 0)),
        ],
        out_shape=[
            jax.ShapeDtypeStruct((N, D), jnp.float32),
            jax.ShapeDtypeStruct((2, D), jnp.float32),
        ],
    )(u1, scu.reshape(1, -1), shu.reshape(1, -1), W, b.reshape(1, -1))


def _u3_body(h_ref, v_ref, sc_ref, sh_ref, ps0_ref, ps1_ref, posq_ref,
             W1a_ref, W1b_ref, h2_ref, pq2_ref, A_ref, B_ref):
    h2 = h_ref[...] + jnp.maximum(v_ref[...] * sc_ref[...] + sh_ref[...], 0.0)
    h2_ref[...] = h2
    psum = ps0_ref[...] + ps1_ref[...]
    deg = psum[:, 3:4]
    pq2 = posq_ref[...][:, :4] + psum / jnp.maximum(deg, 1.0)
    pq2_ref[...] = jnp.pad(pq2, ((0, 0), (0, 12)))
    A_ref[...] = jnp.dot(h2, W1a_ref[...], preferred_element_type=jnp.float32, precision=jax.lax.Precision.HIGHEST)
    B_ref[...] = jnp.dot(h2, W1b_ref[...], preferred_element_type=jnp.float32, precision=jax.lax.Precision.HIGHEST)


def _node_u3(h, v, scv, shv, ps0, ps1, posq, W1a_next, W1b_next):
    N, D = h.shape
    bN = _blk(N, 2048)
    return pl.pallas_call(
        _u3_body,
        grid=(N // bN,),
        in_specs=[
            pl.BlockSpec((bN, D), lambda i: (i, 0)),
            pl.BlockSpec((bN, D), lambda i: (i, 0)),
            pl.BlockSpec((1, D), lambda i: (0, 0)),
            pl.BlockSpec((1, D), lambda i: (0, 0)),
            pl.BlockSpec((bN, 4), lambda i: (i, 0)),
            pl.BlockSpec((bN, 4), lambda i: (i, 0)),
            pl.BlockSpec((bN, 16), lambda i: (i, 0)),
            pl.BlockSpec((D, D), lambda i: (0, 0)),
            pl.BlockSpec((D, D), lambda i: (0, 0)),
        ],
        out_specs=[
            pl.BlockSpec((bN, D), lambda i: (i, 0)),
            pl.BlockSpec((bN, 16), lambda i: (i, 0)),
            pl.BlockSpec((bN, D), lambda i: (i, 0)),
            pl.BlockSpec((bN, D), lambda i: (i, 0)),
        ],
        out_shape=[
            jax.ShapeDtypeStruct((N, D), jnp.float32),
            jax.ShapeDtypeStruct((N, 16), jnp.float32),
            jax.ShapeDtypeStruct((N, D), jnp.float32),
            jax.ShapeDtypeStruct((N, D), jnp.float32),
        ],
    )(h, v, scv.reshape(1, -1), shv.reshape(1, -1), ps0, ps1, posq,
      W1a_next, W1b_next)


def _pred_body(h_ref, wp_ref, bp_ref, out_ref, acc_ref):
    i = pl.program_id(0)

    @pl.when(i == 0)
    def _():
        acc_ref[...] = jnp.zeros_like(acc_ref)

    acc_ref[...] += jnp.sum(h_ref[...], axis=0, keepdims=True)

    @pl.when(i == pl.num_programs(0) - 1)
    def _():
        n = pl.num_programs(0) * h_ref.shape[0]
        pooled = acc_ref[...] / jnp.float32(n)
        out_ref[...] = jnp.dot(pooled, wp_ref[...], preferred_element_type=jnp.float32, precision=jax.lax.Precision.HIGHEST) + bp_ref[...]


def _pred(h, W_pred, b_pred):
    N, D = h.shape
    bN = _blk(N, 2048)
    out = pl.pallas_call(
        _pred_body,
        grid=(N // bN,),
        in_specs=[
            pl.BlockSpec((bN, D), lambda i: (i, 0)),
            pl.BlockSpec((D, 1), lambda i: (0, 0)),
            pl.BlockSpec((1, 1), lambda i: (0, 0)),
        ],
        out_specs=pl.BlockSpec((1, 1), lambda i: (0, 0)),
        out_shape=jax.ShapeDtypeStruct((1, 1), jnp.float32),
        scratch_shapes=[pltpu.VMEM((1, D), jnp.float32)],
    )(h, W_pred, b_pred.reshape(1, 1))
    return out.reshape(-1)


def _sc_gather(A, B, posq, dst3d, src3d, E2):
    N, D = A.shape
    EPW = E2 // (_NC * _NS)
    R = EPW // 1024
    mesh = plsc.VectorSubcoreMesh(core_axis_name="c", subcore_axis_name="s")

    @functools.partial(
        pl.kernel, mesh=mesh,
        compiler_params=pltpu.CompilerParams(use_tc_tiling_on_sc=False),
        out_type=[
            jax.ShapeDtypeStruct((E2, D), jnp.float32),
            jax.ShapeDtypeStruct((E2, D), jnp.float32),
            jax.ShapeDtypeStruct((E2, 16), jnp.float32),
            jax.ShapeDtypeStruct((E2, 16), jnp.float32),
        ],
        scratch_types=[
            pltpu.VMEM((8, _SUB), jnp.int32),
            pltpu.VMEM((8, _SUB), jnp.int32),
            pltpu.VMEM((_RND, D), jnp.float32),
            pltpu.VMEM((_RND, D), jnp.float32),
            pltpu.VMEM((_RND, 16), jnp.float32),
            pltpu.VMEM((_RND, 16), jnp.float32),
            pltpu.SemaphoreType.DMA,
        ],
    )
    def k(A_h, B_h, pq_h, dst_h, src_h, gA_h, gB_h, pD_h, pS_h,
          di, si, bufA, bufB, bufD, bufS, sem):
        c = lax.axis_index("c")
        s = lax.axis_index("s")
        w = s * _NC + c
        base0 = w * EPW

        def round_(r, carry):
            blk = w * R + r
            pltpu.sync_copy(dst_h.at[blk], di)
            pltpu.sync_copy(src_h.at[blk], si)
            for h in range(2):
                base = base0 + r * 1024 + h * _RND
                cps = []
                for g in range(4):
                    row = h * 4 + g
                    sl = pl.ds(g * _SUB, _SUB)
                    cps.append(pltpu.async_copy(A_h.at[di.at[row]], bufA.at[sl, :], sem))
                    cps.append(pltpu.async_copy(B_h.at[si.at[row]], bufB.at[sl, :], sem))
                    cps.append(pltpu.async_copy(pq_h.at[di.at[row]], bufD.at[sl, :], sem))
                    cps.append(pltpu.async_copy(pq_h.at[si.at[row]], bufS.at[sl, :], sem))
                for cp in cps:
                    cp.wait()
                pltpu.sync_copy(bufA, gA_h.at[pl.ds(base, _RND), :])
                pltpu.sync_copy(bufB, gB_h.at[pl.ds(base, _RND), :])
                pltpu.sync_copy(bufD, pD_h.at[pl.ds(base, _RND), :])
                pltpu.sync_copy(bufS, pS_h.at[pl.ds(base, _RND), :])
            return carry

        lax.fori_loop(0, R, round_, 0)

    return k(A, B, posq, dst3d, src3d)


def _sc_scatter_m(mlo, mhi, dst3d, z32, NA):
    E2 = mlo.shape[0]
    DH = mlo.shape[1]
    EPT = E2 // _NS
    R = EPT // 1024
    NPT = NA // _NS
    mesh = plsc.VectorSubcoreMesh(core_axis_name="c", subcore_axis_name="s")

    @functools.partial(
        pl.kernel, mesh=mesh,
        compiler_params=pltpu.CompilerParams(use_tc_tiling_on_sc=False),
        out_type=[
            jax.ShapeDtypeStruct((NA, DH), jnp.float32),
            jax.ShapeDtypeStruct((NA, DH), jnp.float32),
        ],
        scratch_types=[
            pltpu.VMEM((8, _SUB), jnp.int32),
            pltpu.VMEM((_RND, 32), jnp.float32),
            pltpu.VMEM_SHARED((NA, 32), jnp.float32),
            pltpu.SemaphoreType.DMA,
        ],
    )
    def k(mlo_h, mhi_h, dst_h, z32_h, mslo_h, mshi_h, di, mbuf, acc_m, sem):
        c = lax.axis_index("c")
        s = lax.axis_index("s")
        nsl = pl.ds(s * NPT, NPT)
        pltpu.sync_copy(z32_h.at[nsl, :], acc_m.at[nsl, :])
        plsc.subcore_barrier()

        def round_(r, carry):
            blk = s * R + r
            pltpu.sync_copy(dst_h.at[blk], di)
            for h in range(2):
                esl = pl.ds(s * EPT + r * 1024 + h * _RND, _RND)

                @pl.when(c == 0)
                def _():
                    pltpu.sync_copy(mlo_h.at[esl, :], mbuf)

                @pl.when(c == 1)
                def _():
                    pltpu.sync_copy(mhi_h.at[esl, :], mbuf)

                for g in range(4):
                    row = h * 4 + g
                    sl = pl.ds(g * _SUB, _SUB)
                    pltpu.sync_copy(mbuf.at[sl, :], acc_m.at[di.at[row]], add=True)

            return carry

        lax.fori_loop(0, R, round_, 0)
        plsc.subcore_barrier()

        @pl.when(c == 0)
        def _():
            pltpu.sync_copy(acc_m.at[nsl, :], mslo_h.at[nsl, :])

        @pl.when(c == 1)
        def _():
            pltpu.sync_copy(acc_m.at[nsl, :], mshi_h.at[nsl, :])

    return k(mlo, mhi, dst3d, z32)


def _sc_scatter_p(pu, dst3d, z4, NA):
    E2 = pu.shape[0]
    EPT = E2 // (_NC * _NS)
    R = EPT // 1024
    NPT = NA // _NS
    mesh = plsc.VectorSubcoreMesh(core_axis_name="c", subcore_axis_name="s")

    @functools.partial(
        pl.kernel, mesh=mesh,
        compiler_params=pltpu.CompilerParams(use_tc_tiling_on_sc=False),
        out_type=[
            jax.ShapeDtypeStruct((NA, 4), jnp.float32),
            jax.ShapeDtypeStruct((NA, 4), jnp.float32),
        ],
        scratch_types=[
            pltpu.VMEM((8, _SUB), jnp.int32),
            pltpu.VMEM((_RND, 4), jnp.float32),
            pltpu.VMEM_SHARED((NA, 4), jnp.float32),
            pltpu.SemaphoreType.DMA,
        ],
    )
    def k(pu_h, dst_h, z4_h, ps0_h, ps1_h, di, pbuf, acc_p, sem):
        c = lax.axis_index("c")
        s = lax.axis_index("s")
        w = c * _NS + s
        nsl = pl.ds(s * NPT, NPT)
        pltpu.sync_copy(z4_h.at[nsl, :], acc_p.at[nsl, :])
        plsc.subcore_barrier()

        def round_(r, carry):
            blk = w * R + r
            pltpu.sync_copy(dst_h.at[blk], di)
            for h in range(2):
                esl = pl.ds(w * EPT + r * 1024 + h * _RND, _RND)
                pltpu.sync_copy(pu_h.at[esl, :], pbuf)
                for g in range(4):
                    row = h * 4 + g
                    sl = pl.ds(g * _SUB, _SUB)
                    pltpu.sync_copy(pbuf.at[sl, :], acc_p.at[di.at[row]], add=True)

            return carry

        lax.fori_loop(0, R, round_, 0)
        plsc.subcore_barrier()

        @pl.when(c == 0)
        def _():
            pltpu.sync_copy(acc_p.at[nsl, :], ps0_h.at[nsl, :])

        @pl.when(c == 1)
        def _():
            pltpu.sync_copy(acc_p.at[nsl, :], ps1_h.at[nsl, :])

    return k(pu, dst3d, z4)


def _bn_coeffs(stats, count, gamma, beta):
    s, ss = stats[0], stats[1]
    mean = s / count
    var = ss / count - mean * mean
    sc = gamma / jnp.sqrt(var + 1e-5)
    sh = beta - mean * sc
    return sc, sh


def kernel(W_in, b_in, l0_msg_W1, l0_msg_b1, l0_msg_g1, l0_msg_be1, l0_msg_W2, l0_msg_b2, l0_msg_g2, l0_msg_be2, l0_pos_W1, l0_pos_b1, l0_pos_g1, l0_pos_be1, l0_pos_W2, l0_pos_b2, l0_upd_W1, l0_upd_b1, l0_upd_g1, l0_upd_be1, l0_upd_W2, l0_upd_b2, l0_upd_g2, l0_upd_be2, l1_msg_W1, l1_msg_b1, l1_msg_g1, l1_msg_be1, l1_msg_W2, l1_msg_b2, l1_msg_g2, l1_msg_be2, l1_pos_W1, l1_pos_b1, l1_pos_g1, l1_pos_be1, l1_pos_W2, l1_pos_b2, l1_upd_W1, l1_upd_b1, l1_upd_g1, l1_upd_be1, l1_upd_W2, l1_upd_b2, l1_upd_g2, l1_upd_be2, W_pred, b_pred, x, pos, edge_index, edge_attr, batch):
    N = x.shape[0]
    E = edge_index.shape[1]
    D = W_in.shape[1]
    grain = _NC * _NS * 1024
    E2 = ((E + grain - 1) // grain) * grain
    NA = ((N + 1 + 127) // 128) * 128
    src = jnp.concatenate([edge_index[0], jnp.zeros((E2 - E,), jnp.int32)])
    dst = jnp.concatenate([edge_index[1], jnp.full((E2 - E,), N, jnp.int32)])
    src3d = src.reshape(E2 // 1024, 8, _SUB)
    dst3d = dst.reshape(E2 // 1024, 8, _SUB)
    eap = jnp.concatenate([edge_attr, jnp.zeros((E2 - E, 4), jnp.float32)])
    z32 = jnp.zeros((NA, 32), jnp.float32)
    z4 = jnp.zeros((NA, 4), jnp.float32)
    posq = jnp.pad(pos, ((0, 0), (0, 13)))

    L = [
        dict(msg_W1=l0_msg_W1, msg_b1=l0_msg_b1, msg_g1=l0_msg_g1, msg_be1=l0_msg_be1,
             msg_W2=l0_msg_W2, msg_b2=l0_msg_b2, msg_g2=l0_msg_g2, msg_be2=l0_msg_be2,
             pos_W1=l0_pos_W1, pos_b1=l0_pos_b1, pos_g1=l0_pos_g1, pos_be1=l0_pos_be1,
             pos_W2=l0_pos_W2, pos_b2=l0_pos_b2,
             upd_W1=l0_upd_W1, upd_b1=l0_upd_b1, upd_g1=l0_upd_g1, upd_be1=l0_upd_be1,
             upd_W2=l0_upd_W2, upd_b2=l0_upd_b2, upd_g2=l0_upd_g2, upd_be2=l0_upd_be2),
        dict(msg_W1=l1_msg_W1, msg_b1=l1_msg_b1, msg_g1=l1_msg_g1, msg_be1=l1_msg_be1,
             msg_W2=l1_msg_W2, msg_b2=l1_msg_b2, msg_g2=l1_msg_g2, msg_be2=l1_msg_be2,
             pos_W1=l1_pos_W1, pos_b1=l1_pos_b1, pos_g1=l1_pos_g1, pos_be1=l1_pos_be1,
             pos_W2=l1_pos_W2, pos_b2=l1_pos_b2,
             upd_W1=l1_upd_W1, upd_b1=l1_upd_b1, upd_g1=l1_upd_g1, upd_be1=l1_upd_be1,
             upd_W2=l1_upd_W2, upd_b2=l1_upd_b2, upd_g2=l1_upd_g2, upd_be2=l1_upd_be2),
    ]

    W1 = L[0]['msg_W1']
    h, A, B = _node0(x, W_in, b_in, W1[:D], W1[D:2 * D])

    for l in range(2):
        p = L[l]
        W1 = p['msg_W1']
        w1r = W1[2 * D]
        W1e = W1[2 * D + 1:]

        gA, gB, pD, pS = _sc_gather(A, B, posq, dst3d, src3d, E2)

        stats1 = _edge_p1(E, gA, gB, pD, pS, eap, W1e, w1r, p['msg_b1'])
        sc1, sh1 = _bn_coeffs(stats1, E, p['msg_g1'], p['msg_be1'])

        z2, stats2 = _edge_p2(E, gA, gB, pD, pS, eap, W1e, w1r, p['msg_b1'],
                              sc1, sh1, p['msg_W2'], p['msg_b2'])
        sc2, sh2 = _bn_coeffs(stats2, E, p['msg_g2'], p['msg_be2'])

        mlo, mhi, stats3 = _edge_p3(E, z2, sc2, sh2, p['pos_W1'], p['pos_b1'])
        sc3, sh3 = _bn_coeffs(stats3, E, p['pos_g1'], p['pos_be1'])

        pu = _edge_p4(mlo, mhi, pD, pS, sc3, sh3, p['pos_W1'], p['pos_b1'],
                      p['pos_W2'], p['pos_b2'])

        mslo, mshi = _sc_scatter_m(mlo, mhi, dst3d, z32, NA)
        ps0, ps1 = _sc_scatter_p(pu, dst3d, z4, NA)

        Wu = p['upd_W1']
        u1, statsU = _node_u1(h, mslo, mshi, Wu[:D], Wu[D:D + D // 2],
                              Wu[D + D // 2:], p['upd_b1'])
        scu, shu = _bn_coeffs(statsU, N, p['upd_g1'], p['upd_be1'])

        v, statsV = _node_u2(u1, scu, shu, p['upd_W2'], p['upd_b2'])
        scv, shv = _bn_coeffs(statsV, N, p['upd_g2'], p['upd_be2'])

        if l + 1 < 2:
            Wn = L[l + 1]['msg_W1']
            W1a_next, W1b_next = Wn[:D], Wn[D:2 * D]
        else:
            W1a_next = jnp.zeros((D, D), jnp.float32)
            W1b_next = jnp.zeros((D, D), jnp.float32)
        h, posq, A, B = _node_u3(h, v, scv, shv, ps0, ps1, posq,
                                 W1a_next, W1b_next)

    return _pred(h, W_pred, b_pred)


# speedup vs baseline: 2.3638x; 1.1525x over previous
import functools
import math

import jax
import jax.numpy as jnp
from jax import lax
from jax.experimental import pallas as pl
from jax.experimental.pallas import tpu as pltpu
from jax.experimental.pallas import tpu_sc as plsc

_NC = 2
_NS = 16
_SUB = 128
_RND = 512


def _blk(total, cap):
    b = min(total, cap)
    while b > 8:
        if total % b == 0 and b % 8 == 0:
            return b
        b -= 8
    return total


def _node0_body(x_ref, Win_ref, bin_ref, W1a_ref, W1b_ref, h_ref, A_ref, B_ref):
    h = jnp.dot(x_ref[...], Win_ref[...], preferred_element_type=jnp.float32, precision=jax.lax.Precision.HIGHEST) + bin_ref[...]
    h_ref[...] = h
    A_ref[...] = jnp.dot(h, W1a_ref[...], preferred_element_type=jnp.float32, precision=jax.lax.Precision.HIGHEST)
    B_ref[...] = jnp.dot(h, W1b_ref[...], preferred_element_type=jnp.float32, precision=jax.lax.Precision.HIGHEST)


def _node0(x, W_in, b_in, W1a, W1b):
    N, IN = x.shape
    D = W_in.shape[1]
    bN = _blk(N, 2048)
    grid = (N // bN,)
    out = pl.pallas_call(
        _node0_body,
        grid=grid,
        in_specs=[
            pl.BlockSpec((bN, IN), lambda i: (i, 0)),
            pl.BlockSpec((IN, D), lambda i: (0, 0)),
            pl.BlockSpec((1, D), lambda i: (0, 0)),
            pl.BlockSpec((D, D), lambda i: (0, 0)),
            pl.BlockSpec((D, D), lambda i: (0, 0)),
        ],
        out_specs=[
            pl.BlockSpec((bN, D), lambda i: (i, 0)),
            pl.BlockSpec((bN, D), lambda i: (i, 0)),
            pl.BlockSpec((bN, D), lambda i: (i, 0)),
        ],
        out_shape=[
            jax.ShapeDtypeStruct((N, D), jnp.float32),
            jax.ShapeDtypeStruct((N, D), jnp.float32),
            jax.ShapeDtypeStruct((N, D), jnp.float32),
        ],
    )(x, W_in, b_in.reshape(1, -1), W1a, W1b)
    return out


def _accum_stats(stats_ref, z, i, bE, E):
    rows = lax.broadcasted_iota(jnp.int32, (z.shape[0], 1), 0) + i * bE
    msk = (rows < E).astype(jnp.float32)
    zm = z * msk
    s = jnp.sum(zm, axis=0, keepdims=True)
    ss = jnp.sum(zm * z, axis=0, keepdims=True)
    blk = jnp.concatenate([s, ss], axis=0)

    @pl.when(i == 0)
    def _():
        stats_ref[...] = blk

    @pl.when(i > 0)
    def _():
        stats_ref[...] += blk


def _z1_block(gAB_ref, pDS_ref, ea_ref, W1e_ref, w1r_ref, b1_ref):
    D = gAB_ref.shape[1] // 2
    gAB = gAB_ref[...]
    pDS = pDS_ref[...]
    pd = pDS[:, :4] - pDS[:, 16:20]
    rad = jnp.sum(pd * pd, axis=1, keepdims=True)
    return (gAB[:, :D] + gAB[:, D:] + rad * w1r_ref[...]
            + jnp.dot(ea_ref[...], W1e_ref[...], preferred_element_type=jnp.float32, precision=jax.lax.Precision.HIGHEST)
            + b1_ref[...])


def _p1_body(E, gAB_ref, pDS_ref, ea_ref, W1e_ref, w1r_ref, b1_ref, stats_ref):
    i = pl.program_id(0)
    z1 = _z1_block(gAB_ref, pDS_ref, ea_ref, W1e_ref, w1r_ref, b1_ref)
    _accum_stats(stats_ref, z1, i, gAB_ref.shape[0], E)


def _edge_p1(E, gAB, pDS, ea, W1e, w1r, b1):
    E2 = gAB.shape[0]
    D = gAB.shape[1] // 2
    bE = _blk(E2, 4096)
    return pl.pallas_call(
        functools.partial(_p1_body, E),
        grid=(E2 // bE,),
        in_specs=[
            pl.BlockSpec((bE, 2 * D), lambda i: (i, 0)),
            pl.BlockSpec((bE, 32), lambda i: (i, 0)),
            pl.BlockSpec((bE, 4), lambda i: (i, 0)),
            pl.BlockSpec((4, D), lambda i: (0, 0)),
            pl.BlockSpec((1, D), lambda i: (0, 0)),
            pl.BlockSpec((1, D), lambda i: (0, 0)),
        ],
        out_specs=pl.BlockSpec((2, D), lambda i: (0, 0)),
        out_shape=jax.ShapeDtypeStruct((2, D), jnp.float32),
    )(gAB, pDS, ea, W1e, w1r.reshape(1, -1), b1.reshape(1, -1))


def _p2_body(E, gAB_ref, pDS_ref, ea_ref, W1e_ref, w1r_ref, b1_ref,
             sc_ref, sh_ref, W2_ref, b2_ref, z2_ref, stats_ref):
    i = pl.program_id(0)
    z1 = _z1_block(gAB_ref, pDS_ref, ea_ref, W1e_ref, w1r_ref, b1_ref)
    m1 = jnp.maximum(z1 * sc_ref[...] + sh_ref[...], 0.0)
    z2 = jnp.dot(m1, W2_ref[...], preferred_element_type=jnp.float32, precision=jax.lax.Precision.DEFAULT) + b2_ref[...]
    z2_ref[...] = z2
    _accum_stats(stats_ref, z2, i, gAB_ref.shape[0], E)


def _edge_p2(E, gAB, pDS, ea, W1e, w1r, b1, sc1, sh1, W2, b2):
    E2 = gAB.shape[0]
    D = gAB.shape[1] // 2
    bE = _blk(E2, 4096)
    return pl.pallas_call(
        functools.partial(_p2_body, E),
        grid=(E2 // bE,),
        in_specs=[
            pl.BlockSpec((bE, 2 * D), lambda i: (i, 0)),
            pl.BlockSpec((bE, 32), lambda i: (i, 0)),
            pl.BlockSpec((bE, 4), lambda i: (i, 0)),
            pl.BlockSpec((4, D), lambda i: (0, 0)),
            pl.BlockSpec((1, D), lambda i: (0, 0)),
            pl.BlockSpec((1, D), lambda i: (0, 0)),
            pl.BlockSpec((1, D), lambda i: (0, 0)),
            pl.BlockSpec((1, D), lambda i: (0, 0)),
            pl.BlockSpec((D, D), lambda i: (0, 0)),
            pl.BlockSpec((1, D), lambda i: (0, 0)),
        ],
        out_specs=[
            pl.BlockSpec((bE, D), lambda i: (i, 0)),
            pl.BlockSpec((2, D), lambda i: (0, 0)),
        ],
        out_shape=[
            jax.ShapeDtypeStruct((E2, D), jnp.float32),
            jax.ShapeDtypeStruct((2, D), jnp.float32),
        ],
    )(gAB, pDS, ea, W1e, w1r.reshape(1, -1), b1.reshape(1, -1),
      sc1.reshape(1, -1), sh1.reshape(1, -1), W2, b2.reshape(1, -1))


def _p3_body(E, z2_ref, sc_ref, sh_ref, W3_ref, b3_ref, mlo_ref, mhi_ref, stats_ref):
    i = pl.program_id(0)
    m = jnp.maximum(z2_ref[...] * sc_ref[...] + sh_ref[...], 0.0)
    DH = mlo_ref.shape[1]
    mlo_ref[...] = m[:, :DH]
    mhi_ref[...] = m[:, DH:]
    z3 = jnp.dot(m, W3_ref[...], preferred_element_type=jnp.float32, precision=jax.lax.Precision.DEFAULT) + b3_ref[...]
    _accum_stats(stats_ref, z3, i, z2_ref.shape[0], E)


def _edge_p3(E, z2, sc2, sh2, W3, b3):
    E2, D = z2.shape
    DH = D // 2
    bE = _blk(E2, 4096)
    return pl.pallas_call(
        functools.partial(_p3_body, E),
        grid=(E2 // bE,),
        in_specs=[
            pl.BlockSpec((bE, D), lambda i: (i, 0)),
            pl.BlockSpec((1, D), lambda i: (0, 0)),
            pl.BlockSpec((1, D), lambda i: (0, 0)),
            pl.BlockSpec((D, D), lambda i: (0, 0)),
            pl.BlockSpec((1, D), lambda i: (0, 0)),
        ],
        out_specs=[
            pl.BlockSpec((bE, DH), lambda i: (i, 0)),
            pl.BlockSpec((bE, DH), lambda i: (i, 0)),
            pl.BlockSpec((2, D), lambda i: (0, 0)),
        ],
        out_shape=[
            jax.ShapeDtypeStruct((E2, DH), jnp.float32),
            jax.ShapeDtypeStruct((E2, DH), jnp.float32),
            jax.ShapeDtypeStruct((2, D), jnp.float32),
        ],
    )(z2, sc2.reshape(1, -1), sh2.reshape(1, -1), W3, b3.reshape(1, -1))


def _p4_body(mlo_ref, mhi_ref, pDS_ref, sc_ref, sh_ref, W3_ref, b3_ref, W4_ref, b4_ref, pu_ref):
    m = jnp.concatenate([mlo_ref[...], mhi_ref[...]], axis=1)
    z3 = jnp.dot(m, W3_ref[...], preferred_element_type=jnp.float32, precision=jax.lax.Precision.DEFAULT) + b3_ref[...]
    pm1 = jnp.maximum(z3 * sc_ref[...] + sh_ref[...], 0.0)
    pm = jnp.dot(pm1, W4_ref[...], preferred_element_type=jnp.float32, precision=jax.lax.Precision.DEFAULT) + b4_ref[...]
    pDS = pDS_ref[...]
    pu = (pDS[:, :4] - pDS[:, 16:20]) * pm
    lane = lax.broadcasted_iota(jnp.int32, pu.shape, 1)
    pu_ref[...] = jnp.where(lane == 3, 1.0, pu)


def _edge_p4(mlo, mhi, pDS, sc3, sh3, W3, b3, W4, b4):
    E2, DH = mlo.shape
    D = 2 * DH
    bE = _blk(E2, 4096)
    return pl.pallas_call(
        _p4_body,
        grid=(E2 // bE,),
        in_specs=[
            pl.BlockSpec((bE, DH), lambda i: (i, 0)),
            pl.BlockSpec((bE, DH), lambda i: (i, 0)),
            pl.BlockSpec((bE, 32), lambda i: (i, 0)),
            pl.BlockSpec((1, D), lambda i: (0, 0)),
            pl.BlockSpec((1, D), lambda i: (0, 0)),
            pl.BlockSpec((D, D), lambda i: (0, 0)),
            pl.BlockSpec((1, D), lambda i: (0, 0)),
            pl.BlockSpec((D, 1), lambda i: (0, 0)),
            pl.BlockSpec((1, 1), lambda i: (0, 0)),
        ],
        out_specs=pl.BlockSpec((bE, 4), lambda i: (i, 0)),
        out_shape=jax.ShapeDtypeStruct((E2, 4), jnp.float32),
    )(mlo, mhi, pDS, sc3.reshape(1, -1), sh3.reshape(1, -1), W3, b3.reshape(1, -1),
      W4, b4.reshape(1, 1))


def _u1_body(h_ref, mlo_ref, mhi_ref, Wa_ref, Wblo_ref, Wbhi_ref, b_ref, u1_ref, stats_ref):
    i = pl.program_id(0)
    u1 = (jnp.dot(h_ref[...], Wa_ref[...], preferred_element_type=jnp.float32, precision=jax.lax.Precision.HIGHEST)
          + jnp.dot(mlo_ref[...], Wblo_ref[...], preferred_element_type=jnp.float32, precision=jax.lax.Precision.HIGHEST)
          + jnp.dot(mhi_ref[...], Wbhi_ref[...], preferred_element_type=jnp.float32, precision=jax.lax.Precision.HIGHEST)
          + b_ref[...])
    u1_ref[...] = u1
    _accum_stats(stats_ref, u1, i, u1.shape[0], u1.shape[0] * pl.num_programs(0))


def _node_u1(h, mslo, mshi, Wa, Wblo, Wbhi, b):
    N, D = h.shape
    DH = mslo.shape[1]
    bN = _blk(N, 2048)
    return pl.pallas_call(
        _u1_body,
        grid=(N // bN,),
        in_specs=[
            pl.BlockSpec((bN, D), lambda i: (i, 0)),
            pl.BlockSpec((bN, DH), lambda i: (i, 0)),
            pl.BlockSpec((bN, DH), lambda i: (i, 0)),
            pl.BlockSpec((D, D), lambda i: (0, 0)),
            pl.BlockSpec((DH, D), lambda i: (0, 0)),
            pl.BlockSpec((DH, D), lambda i: (0, 0)),
            pl.BlockSpec((1, D), lambda i: (0, 0)),
        ],
        out_specs=[
            pl.BlockSpec((bN, D), lambda i: (i, 0)),
            pl.BlockSpec((2, D), lambda i: (0, 0)),
        ],
        out_shape=[
            jax.ShapeDtypeStruct((N, D), jnp.float32),
            jax.ShapeDtypeStruct((2, D), jnp.float32),
        ],
    )(h, mslo, mshi, Wa, Wblo, Wbhi, b.reshape(1, -1))


def _u2_body(u1_ref, sc_ref, sh_ref, W_ref, b_ref, v_ref, stats_ref):
    i = pl.program_id(0)
    r = jnp.maximum(u1_ref[...] * sc_ref[...] + sh_ref[...], 0.0)
    v = jnp.dot(r, W_ref[...], preferred_element_type=jnp.float32, precision=jax.lax.Precision.HIGHEST) + b_ref[...]
    v_ref[...] = v
    _accum_stats(stats_ref, v, i, v.shape[0], v.shape[0] * pl.num_programs(0))


def _node_u2(u1, scu, shu, W, b):
    N, D = u1.shape
    bN = _blk(N, 2048)
    return pl.pallas_call(
        _u2_body,
        grid=(N // bN,),
        in_specs=[
            pl.BlockSpec((bN, D), lambda i: (i, 0)),
            pl.BlockSpec((1, D), lambda i: (0, 0)),
            pl.BlockSpec((1, D), lambda i: (0, 0)),
            pl.BlockSpec((D, D), lambda i: (0, 0)),
            pl.BlockSpec((1, D), lambda i: (0, 0)),
        ],
        out_specs=[
            pl.BlockSpec((bN, D), lambda i: (i, 0)),
            pl.BlockSpec((2, D), lambda i: (0, 0)),
        ],
        out_shape=[
            jax.ShapeDtypeStruct((N, D), jnp.float32),
            jax.ShapeDtypeStruct((2, D), jnp.float32),
        ],
    )(u1, scu.reshape(1, -1), shu.reshape(1, -1), W, b.reshape(1, -1))


def _u3_body(h_ref, v_ref, sc_ref, sh_ref, ps0_ref, ps1_ref, posq_ref,
             W1a_ref, W1b_ref, h2_ref, pq2_ref, A_ref, B_ref):
    h2 = h_ref[...] + jnp.maximum(v_ref[...] * sc_ref[...] + sh_ref[...], 0.0)
    h2_ref[...] = h2
    psum = ps0_ref[...] + ps1_ref[...]
    deg = psum[:, 3:4]
    pq2 = posq_ref[...][:, :4] + psum / jnp.maximum(deg, 1.0)
    pq2_ref[...] = jnp.pad(pq2, ((0, 0), (0, 12)))
    A_ref[...] = jnp.dot(h2, W1a_ref[...], preferred_element_type=jnp.float32, precision=jax.lax.Precision.HIGHEST)
    B_ref[...] = jnp.dot(h2, W1b_ref[...], preferred_element_type=jnp.float32, precision=jax.lax.Precision.HIGHEST)


def _node_u3(h, v, scv, shv, ps0, ps1, posq, W1a_next, W1b_next):
    N, D = h.shape
    bN = _blk(N, 2048)
    return pl.pallas_call(
        _u3_body,
        grid=(N // bN,),
        in_specs=[
            pl.BlockSpec((bN, D), lambda i: (i, 0)),
            pl.BlockSpec((bN, D), lambda i: (i, 0)),
            pl.BlockSpec((1, D), lambda i: (0, 0)),
            pl.BlockSpec((1, D), lambda i: (0, 0)),
            pl.BlockSpec((bN, 4), lambda i: (i, 0)),
            pl.BlockSpec((bN, 4), lambda i: (i, 0)),
            pl.BlockSpec((bN, 16), lambda i: (i, 0)),
            pl.BlockSpec((D, D), lambda i: (0, 0)),
            pl.BlockSpec((D, D), lambda i: (0, 0)),
        ],
        out_specs=[
            pl.BlockSpec((bN, D), lambda i: (i, 0)),
            pl.BlockSpec((bN, 16), lambda i: (i, 0)),
            pl.BlockSpec((bN, D), lambda i: (i, 0)),
            pl.BlockSpec((bN, D), lambda i: (i, 0)),
        ],
        out_shape=[
            jax.ShapeDtypeStruct((N, D), jnp.float32),
            jax.ShapeDtypeStruct((N, 16), jnp.float32),
            jax.ShapeDtypeStruct((N, D), jnp.float32),
            jax.ShapeDtypeStruct((N, D), jnp.float32),
        ],
    )(h, v, scv.reshape(1, -1), shv.reshape(1, -1), ps0, ps1, posq,
      W1a_next, W1b_next)


def _pred_body(h_ref, wp_ref, bp_ref, out_ref, acc_ref):
    i = pl.program_id(0)

    @pl.when(i == 0)
    def _():
        acc_ref[...] = jnp.zeros_like(acc_ref)

    acc_ref[...] += jnp.sum(h_ref[...], axis=0, keepdims=True)

    @pl.when(i == pl.num_programs(0) - 1)
    def _():
        n = pl.num_programs(0) * h_ref.shape[0]
        pooled = acc_ref[...] / jnp.float32(n)
        out_ref[...] = jnp.dot(pooled, wp_ref[...], preferred_element_type=jnp.float32, precision=jax.lax.Precision.HIGHEST) + bp_ref[...]


def _pred(h, W_pred, b_pred):
    N, D = h.shape
    bN = _blk(N, 2048)
    out = pl.pallas_call(
        _pred_body,
        grid=(N // bN,),
        in_specs=[
            pl.BlockSpec((bN, D), lambda i: (i, 0)),
            pl.BlockSpec((D, 1), lambda i: (0, 0)),
            pl.BlockSpec((1, 1), lambda i: (0, 0)),
        ],
        out_specs=pl.BlockSpec((1, 1), lambda i: (0, 0)),
        out_shape=jax.ShapeDtypeStruct((1, 1), jnp.float32),
        scratch_shapes=[pltpu.VMEM((1, D), jnp.float32)],
    )(h, W_pred, b_pred.reshape(1, 1))
    return out.reshape(-1)


def _sc_gather(A, B, posq, dst3d, src3d, E2):
    N, D = A.shape
    EPW = E2 // (_NC * _NS)
    R = EPW // 1024
    mesh = plsc.VectorSubcoreMesh(core_axis_name="c", subcore_axis_name="s")

    @functools.partial(
        pl.kernel, mesh=mesh,
        compiler_params=pltpu.CompilerParams(use_tc_tiling_on_sc=False),
        out_type=[
            jax.ShapeDtypeStruct((E2, 2 * D), jnp.float32),
            jax.ShapeDtypeStruct((E2, 32), jnp.float32),
        ],
        scratch_types=[
            pltpu.VMEM((8, _SUB), jnp.int32),
            pltpu.VMEM((8, _SUB), jnp.int32),
            pltpu.VMEM((_RND, D), jnp.float32),
            pltpu.VMEM((_RND, D), jnp.float32),
            pltpu.VMEM((_RND, 16), jnp.float32),
            pltpu.VMEM((_RND, 16), jnp.float32),
            pltpu.SemaphoreType.DMA,
        ],
    )
    def k(A_h, B_h, pq_h, dst_h, src_h, gAB_h, pDS_h,
          di, si, bufA, bufB, bufD, bufS, sem):
        c = lax.axis_index("c")
        s = lax.axis_index("s")
        w = s * _NC + c
        base0 = w * EPW

        def round_(r, carry):
            blk = w * R + r
            pltpu.sync_copy(dst_h.at[blk], di)
            pltpu.sync_copy(src_h.at[blk], si)
            for h in range(2):
                base = base0 + r * 1024 + h * _RND
                cps = []
                for g in range(4):
                    row = h * 4 + g
                    sl = pl.ds(g * _SUB, _SUB)
                    cps.append(pltpu.async_copy(A_h.at[di.at[row]], bufA.at[sl, :], sem))
                    cps.append(pltpu.async_copy(B_h.at[si.at[row]], bufB.at[sl, :], sem))
                    cps.append(pltpu.async_copy(pq_h.at[di.at[row]], bufD.at[sl, :], sem))
                    cps.append(pltpu.async_copy(pq_h.at[si.at[row]], bufS.at[sl, :], sem))
                for cp in cps:
                    cp.wait()
                pltpu.sync_copy(bufA, gAB_h.at[pl.ds(base, _RND), pl.ds(0, D)])
                pltpu.sync_copy(bufB, gAB_h.at[pl.ds(base, _RND), pl.ds(D, D)])
                pltpu.sync_copy(bufD, pDS_h.at[pl.ds(base, _RND), pl.ds(0, 16)])
                pltpu.sync_copy(bufS, pDS_h.at[pl.ds(base, _RND), pl.ds(16, 16)])
            return carry

        lax.fori_loop(0, R, round_, 0)

    return k(A, B, posq, dst3d, src3d)


def _sc_scatter_m(mlo, mhi, dst3d, z32, NA):
    E2 = mlo.shape[0]
    DH = mlo.shape[1]
    EPT = E2 // _NS
    R = EPT // 1024
    NPT = NA // _NS
    mesh = plsc.VectorSubcoreMesh(core_axis_name="c", subcore_axis_name="s")

    @functools.partial(
        pl.kernel, mesh=mesh,
        compiler_params=pltpu.CompilerParams(use_tc_tiling_on_sc=False),
        out_type=[
            jax.ShapeDtypeStruct((NA, DH), jnp.float32),
            jax.ShapeDtypeStruct((NA, DH), jnp.float32),
        ],
        scratch_types=[
            pltpu.VMEM((8, _SUB), jnp.int32),
            pltpu.VMEM((_RND, 32), jnp.float32),
            pltpu.VMEM_SHARED((NA, 32), jnp.float32),
            pltpu.SemaphoreType.DMA,
        ],
    )
    def k(mlo_h, mhi_h, dst_h, z32_h, mslo_h, mshi_h, di, mbuf, acc_m, sem):
        c = lax.axis_index("c")
        s = lax.axis_index("s")
        nsl = pl.ds(s * NPT, NPT)
        pltpu.sync_copy(z32_h.at[nsl, :], acc_m.at[nsl, :])
        plsc.subcore_barrier()

        def round_(r, carry):
            blk = s * R + r
            pltpu.sync_copy(dst_h.at[blk], di)
            for h in range(2):
                esl = pl.ds(s * EPT + r * 1024 + h * _RND, _RND)

                @pl.when(c == 0)
                def _():
                    pltpu.sync_copy(mlo_h.at[esl, :], mbuf)

                @pl.when(c == 1)
                def _():
                    pltpu.sync_copy(mhi_h.at[esl, :], mbuf)

                for g in range(4):
                    row = h * 4 + g
                    sl = pl.ds(g * _SUB, _SUB)
                    pltpu.sync_copy(mbuf.at[sl, :], acc_m.at[di.at[row]], add=True)

            return carry

        lax.fori_loop(0, R, round_, 0)
        plsc.subcore_barrier()

        @pl.when(c == 0)
        def _():
            pltpu.sync_copy(acc_m.at[nsl, :], mslo_h.at[nsl, :])

        @pl.when(c == 1)
        def _():
            pltpu.sync_copy(acc_m.at[nsl, :], mshi_h.at[nsl, :])

    return k(mlo, mhi, dst3d, z32)


def _sc_scatter_p(pu, dst3d, z4, NA):
    E2 = pu.shape[0]
    EPT = E2 // (_NC * _NS)
    R = EPT // 1024
    NPT = NA // _NS
    mesh = plsc.VectorSubcoreMesh(core_axis_name="c", subcore_axis_name="s")

    @functools.partial(
        pl.kernel, mesh=mesh,
        compiler_params=pltpu.CompilerParams(use_tc_tiling_on_sc=False),
        out_type=[
            jax.ShapeDtypeStruct((NA, 4), jnp.float32),
            jax.ShapeDtypeStruct((NA, 4), jnp.float32),
        ],
        scratch_types=[
            pltpu.VMEM((8, _SUB), jnp.int32),
            pltpu.VMEM((_RND, 4), jnp.float32),
            pltpu.VMEM_SHARED((NA, 4), jnp.float32),
            pltpu.SemaphoreType.DMA,
        ],
    )
    def k(pu_h, dst_h, z4_h, ps0_h, ps1_h, di, pbuf, acc_p, sem):
        c = lax.axis_index("c")
        s = lax.axis_index("s")
        w = c * _NS + s
        nsl = pl.ds(s * NPT, NPT)
        pltpu.sync_copy(z4_h.at[nsl, :], acc_p.at[nsl, :])
        plsc.subcore_barrier()

        def round_(r, carry):
            blk = w * R + r
            pltpu.sync_copy(dst_h.at[blk], di)
            for h in range(2):
                esl = pl.ds(w * EPT + r * 1024 + h * _RND, _RND)
                pltpu.sync_copy(pu_h.at[esl, :], pbuf)
                for g in range(4):
                    row = h * 4 + g
                    sl = pl.ds(g * _SUB, _SUB)
                    pltpu.sync_copy(pbuf.at[sl, :], acc_p.at[di.at[row]], add=True)

            return carry

        lax.fori_loop(0, R, round_, 0)
        plsc.subcore_barrier()

        @pl.when(c == 0)
        def _():
            pltpu.sync_copy(acc_p.at[nsl, :], ps0_h.at[nsl, :])

        @pl.when(c == 1)
        def _():
            pltpu.sync_copy(acc_p.at[nsl, :], ps1_h.at[nsl, :])

    return k(pu, dst3d, z4)


def _bn_coeffs(stats, count, gamma, beta):
    s, ss = stats[0], stats[1]
    mean = s / count
    var = ss / count - mean * mean
    sc = gamma / jnp.sqrt(var + 1e-5)
    sh = beta - mean * sc
    return sc, sh


def kernel(W_in, b_in, l0_msg_W1, l0_msg_b1, l0_msg_g1, l0_msg_be1, l0_msg_W2, l0_msg_b2, l0_msg_g2, l0_msg_be2, l0_pos_W1, l0_pos_b1, l0_pos_g1, l0_pos_be1, l0_pos_W2, l0_pos_b2, l0_upd_W1, l0_upd_b1, l0_upd_g1, l0_upd_be1, l0_upd_W2, l0_upd_b2, l0_upd_g2, l0_upd_be2, l1_msg_W1, l1_msg_b1, l1_msg_g1, l1_msg_be1, l1_msg_W2, l1_msg_b2, l1_msg_g2, l1_msg_be2, l1_pos_W1, l1_pos_b1, l1_pos_g1, l1_pos_be1, l1_pos_W2, l1_pos_b2, l1_upd_W1, l1_upd_b1, l1_upd_g1, l1_upd_be1, l1_upd_W2, l1_upd_b2, l1_upd_g2, l1_upd_be2, W_pred, b_pred, x, pos, edge_index, edge_attr, batch):
    N = x.shape[0]
    E = edge_index.shape[1]
    D = W_in.shape[1]
    grain = _NC * _NS * 1024
    E2 = ((E + grain - 1) // grain) * grain
    NA = ((N + 1 + 127) // 128) * 128
    src = jnp.concatenate([edge_index[0], jnp.zeros((E2 - E,), jnp.int32)])
    dst = jnp.concatenate([edge_index[1], jnp.full((E2 - E,), N, jnp.int32)])
    src3d = src.reshape(E2 // 1024, 8, _SUB)
    dst3d = dst.reshape(E2 // 1024, 8, _SUB)
    eap = jnp.concatenate([edge_attr, jnp.zeros((E2 - E, 4), jnp.float32)])
    z32 = jnp.zeros((NA, 32), jnp.float32)
    z4 = jnp.zeros((NA, 4), jnp.float32)
    posq = jnp.pad(pos, ((0, 0), (0, 13)))

    L = [
        dict(msg_W1=l0_msg_W1, msg_b1=l0_msg_b1, msg_g1=l0_msg_g1, msg_be1=l0_msg_be1,
             msg_W2=l0_msg_W2, msg_b2=l0_msg_b2, msg_g2=l0_msg_g2, msg_be2=l0_msg_be2,
             pos_W1=l0_pos_W1, pos_b1=l0_pos_b1, pos_g1=l0_pos_g1, pos_be1=l0_pos_be1,
             pos_W2=l0_pos_W2, pos_b2=l0_pos_b2,
             upd_W1=l0_upd_W1, upd_b1=l0_upd_b1, upd_g1=l0_upd_g1, upd_be1=l0_upd_be1,
             upd_W2=l0_upd_W2, upd_b2=l0_upd_b2, upd_g2=l0_upd_g2, upd_be2=l0_upd_be2),
        dict(msg_W1=l1_msg_W1, msg_b1=l1_msg_b1, msg_g1=l1_msg_g1, msg_be1=l1_msg_be1,
             msg_W2=l1_msg_W2, msg_b2=l1_msg_b2, msg_g2=l1_msg_g2, msg_be2=l1_msg_be2,
             pos_W1=l1_pos_W1, pos_b1=l1_pos_b1, pos_g1=l1_pos_g1, pos_be1=l1_pos_be1,
             pos_W2=l1_pos_W2, pos_b2=l1_pos_b2,
             upd_W1=l1_upd_W1, upd_b1=l1_upd_b1, upd_g1=l1_upd_g1, upd_be1=l1_upd_be1,
             upd_W2=l1_upd_W2, upd_b2=l1_upd_b2, upd_g2=l1_upd_g2, upd_be2=l1_upd_be2),
    ]

    W1 = L[0]['msg_W1']
    h, A, B = _node0(x, W_in, b_in, W1[:D], W1[D:2 * D])

    for l in range(2):
        p = L[l]
        W1 = p['msg_W1']
        w1r = W1[2 * D]
        W1e = W1[2 * D + 1:]

        gAB, pDS = _sc_gather(A, B, posq, dst3d, src3d, E2)

        stats1 = _edge_p1(E, gAB, pDS, eap, W1e, w1r, p['msg_b1'])
        sc1, sh1 = _bn_coeffs(stats1, E, p['msg_g1'], p['msg_be1'])

        z2, stats2 = _edge_p2(E, gAB, pDS, eap, W1e, w1r, p['msg_b1'],
                              sc1, sh1, p['msg_W2'], p['msg_b2'])
        sc2, sh2 = _bn_coeffs(stats2, E, p['msg_g2'], p['msg_be2'])

        mlo, mhi, stats3 = _edge_p3(E, z2, sc2, sh2, p['pos_W1'], p['pos_b1'])
        sc3, sh3 = _bn_coeffs(stats3, E, p['pos_g1'], p['pos_be1'])

        pu = _edge_p4(mlo, mhi, pDS, sc3, sh3, p['pos_W1'], p['pos_b1'],
                      p['pos_W2'], p['pos_b2'])

        mslo, mshi = _sc_scatter_m(mlo, mhi, dst3d, z32, NA)
        ps0, ps1 = _sc_scatter_p(pu, dst3d, z4, NA)

        Wu = p['upd_W1']
        u1, statsU = _node_u1(h, mslo, mshi, Wu[:D], Wu[D:D + D // 2],
                              Wu[D + D // 2:], p['upd_b1'])
        scu, shu = _bn_coeffs(statsU, N, p['upd_g1'], p['upd_be1'])

        v, statsV = _node_u2(u1, scu, shu, p['upd_W2'], p['upd_b2'])
        scv, shv = _bn_coeffs(statsV, N, p['upd_g2'], p['upd_be2'])

        if l + 1 < 2:
            Wn = L[l + 1]['msg_W1']
            W1a_next, W1b_next = Wn[:D], Wn[D:2 * D]
        else:
            W1a_next = jnp.zeros((D, D), jnp.float32)
            W1b_next = jnp.zeros((D, D), jnp.float32)
        h, posq, A, B = _node_u3(h, v, scv, shv, ps0, ps1, posq,
                                 W1a_next, W1b_next)

    return _pred(h, W_pred, b_pred)
